# Initial kernel scaffold; baseline (speedup 1.0000x reference)
import jax
import jax.numpy as jnp
from jax.experimental import pallas as pl


def kernel(x, adj_t, Wl0, Wr0, b0, Wl1, Wr1, b1, Wl2, Wr2, b2):
    raise NotImplementedError("write your pallas kernel here")


# speedup vs baseline: 7.2679x; 7.2679x over previous
import functools

import jax
import jax.numpy as jnp
from jax import lax
from jax.experimental import pallas as pl
from jax.experimental.pallas import tpu as pltpu
from jax.experimental.pallas import tpu_sc as plsc

N = 10000
E = 320000
NC = 2
NS = 16
NW = NC * NS
ET = E // NW
K = 80
NCHUNK = ET // K
RPT = 624
RPT_LAST = N - RPT * (NS - 1)


def _sc_segment_sum(y, src2d, dst2d, zrows, zdeg, with_deg):
  C = y.shape[1]
  mesh = plsc.VectorSubcoreMesh(core_axis_name="c", subcore_axis_name="s")
  out_type = [jax.ShapeDtypeStruct((NC, N, C), jnp.float32)]
  scratch = [
      pltpu.VMEM_SHARED((N, C), jnp.float32),
      pltpu.VMEM((NCHUNK, K), jnp.int32),
      pltpu.VMEM((NCHUNK, K), jnp.int32),
      pltpu.VMEM((K, C), jnp.float32),
      pltpu.SemaphoreType.DMA,
  ]
  if with_deg:
    out_type.append(jax.ShapeDtypeStruct((NC, N), jnp.float32))
    scratch += [
        pltpu.VMEM_SHARED((N,), jnp.float32),
        pltpu.VMEM((K,), jnp.float32),
    ]

  def body(y_h, s_h, d_h, zr_h, zd_h, *rest):
    if with_deg:
      out_h, deg_h, acc, sidx, didx, rows, sem, dacc, ones = rest
    else:
      out_h, acc, sidx, didx, rows, sem = rest
    cid = lax.axis_index("c")
    sid = lax.axis_index("s")
    wid = cid * NS + sid
    r0 = pl.multiple_of(sid * RPT, 8)
    @pl.when(sid < NS - 1)
    def _():
      pltpu.sync_copy(zr_h.at[pl.ds(0, RPT)], acc.at[pl.ds(r0, RPT)])
    @pl.when(sid == NS - 1)
    def _():
      pltpu.sync_copy(zr_h, acc.at[pl.ds(RPT * (NS - 1), RPT_LAST)])
    if with_deg:
      @pl.when(sid == 0)
      def _():
        pltpu.sync_copy(zd_h, dacc)
      for j in range(K // 16):
        ones[pl.ds(j * 16, 16)] = jnp.ones((16,), jnp.float32)
    pltpu.sync_copy(s_h.at[wid], sidx)
    pltpu.sync_copy(d_h.at[wid], didx)
    plsc.subcore_barrier()

    def step(i, carry):
      pltpu.async_copy(y_h.at[sidx.at[i]], rows, sem).wait()
      pltpu.sync_copy(rows, acc.at[didx.at[i]], add=True)
      if with_deg:
        pltpu.sync_copy(ones, dacc.at[didx.at[i]], add=True)
      return carry

    lax.fori_loop(0, NCHUNK, step, 0)
    plsc.subcore_barrier()
    @pl.when(sid < NS - 1)
    def _():
      pltpu.sync_copy(acc.at[pl.ds(r0, RPT)], out_h.at[cid, pl.ds(r0, RPT)])
    @pl.when(sid == NS - 1)
    def _():
      last = RPT * (NS - 1)
      pltpu.sync_copy(acc.at[pl.ds(last, RPT_LAST)],
                      out_h.at[cid, pl.ds(last, RPT_LAST)])
    if with_deg:
      @pl.when(sid == 0)
      def _():
        pltpu.sync_copy(dacc, deg_h.at[cid])

  fn = pl.kernel(body, out_type=out_type, mesh=mesh, scratch_types=scratch)
  return fn(y, src2d, dst2d, zrows, zdeg)


ROWS_B = 1000


def _tc_matmul2(x, Wl, Wr, b):
  Co = Wl.shape[0]
  Cz = Wr.shape[0]

  def body(x_ref, wl_ref, wr_ref, b_ref, y_ref, z_ref):
    xb = x_ref[...]
    dn = (((1,), (1,)), ((), ()))
    y_ref[...] = lax.dot_general(xb, wl_ref[...], dn,
                                 preferred_element_type=jnp.float32)
    z_ref[...] = lax.dot_general(xb, wr_ref[...], dn,
                                 preferred_element_type=jnp.float32) + b_ref[...]

  grid = (N // ROWS_B,)
  return pl.pallas_call(
      body,
      grid=grid,
      in_specs=[
          pl.BlockSpec((ROWS_B, x.shape[1]), lambda i: (i, 0)),
          pl.BlockSpec(Wl.shape, lambda i: (0, 0)),
          pl.BlockSpec(Wr.shape, lambda i: (0, 0)),
          pl.BlockSpec((1, Cz), lambda i: (0, 0)),
      ],
      out_specs=[
          pl.BlockSpec((ROWS_B, Co), lambda i: (i, 0)),
          pl.BlockSpec((ROWS_B, Cz), lambda i: (i, 0)),
      ],
      out_shape=[
          jax.ShapeDtypeStruct((N, Co), jnp.float32),
          jax.ShapeDtypeStruct((N, Cz), jnp.float32),
      ],
  )(x, Wl, Wr, b)


def _tc_combine_matmul2(p, z, degp, Wl, Wr, b):
  C = z.shape[1]
  Co = Wl.shape[0]
  Cz = Wr.shape[0]

  def body(p_ref, z_ref, d_ref, wl_ref, wr_ref, b_ref, y_ref, z2_ref):
    pb = p_ref[...]
    deg = d_ref[0] + d_ref[1]
    dinv = 1.0 / jnp.maximum(deg, 1.0)
    h = jnp.maximum((pb[0] + pb[1]) * dinv + z_ref[...], 0.0)
    dn = (((1,), (1,)), ((), ()))
    y_ref[...] = lax.dot_general(h, wl_ref[...], dn,
                                 preferred_element_type=jnp.float32)
    z2_ref[...] = lax.dot_general(h, wr_ref[...], dn,
                                  preferred_element_type=jnp.float32) + b_ref[...]

  grid = (N // ROWS_B,)
  return pl.pallas_call(
      body,
      grid=grid,
      in_specs=[
          pl.BlockSpec((NC, ROWS_B, C), lambda i: (0, i, 0)),
          pl.BlockSpec((ROWS_B, C), lambda i: (i, 0)),
          pl.BlockSpec((NC, ROWS_B, 1), lambda i: (0, i, 0)),
          pl.BlockSpec(Wl.shape, lambda i: (0, 0)),
          pl.BlockSpec(Wr.shape, lambda i: (0, 0)),
          pl.BlockSpec((1, Cz), lambda i: (0, 0)),
      ],
      out_specs=[
          pl.BlockSpec((ROWS_B, Co), lambda i: (i, 0)),
          pl.BlockSpec((ROWS_B, Cz), lambda i: (i, 0)),
      ],
      out_shape=[
          jax.ShapeDtypeStruct((N, Co), jnp.float32),
          jax.ShapeDtypeStruct((N, Cz), jnp.float32),
      ],
  )(p, z, degp, Wl, Wr, b)


def _tc_combine_hz(p, z, degp, Wr, b):
  C = z.shape[1]
  Cz = Wr.shape[0]

  def body(p_ref, z_ref, d_ref, wr_ref, b_ref, h_ref, z2_ref):
    pb = p_ref[...]
    deg = d_ref[0] + d_ref[1]
    dinv = 1.0 / jnp.maximum(deg, 1.0)
    h = jnp.maximum((pb[0] + pb[1]) * dinv + z_ref[...], 0.0)
    h_ref[...] = h
    dn = (((1,), (1,)), ((), ()))
    z2_ref[...] = lax.dot_general(h, wr_ref[...], dn,
                                  preferred_element_type=jnp.float32) + b_ref[...]

  grid = (N // ROWS_B,)
  return pl.pallas_call(
      body,
      grid=grid,
      in_specs=[
          pl.BlockSpec((NC, ROWS_B, C), lambda i: (0, i, 0)),
          pl.BlockSpec((ROWS_B, C), lambda i: (i, 0)),
          pl.BlockSpec((NC, ROWS_B, 1), lambda i: (0, i, 0)),
          pl.BlockSpec(Wr.shape, lambda i: (0, 0)),
          pl.BlockSpec((1, Cz), lambda i: (0, 0)),
      ],
      out_specs=[
          pl.BlockSpec((ROWS_B, C), lambda i: (i, 0)),
          pl.BlockSpec((ROWS_B, Cz), lambda i: (i, 0)),
      ],
      out_shape=[
          jax.ShapeDtypeStruct((N, C), jnp.float32),
          jax.ShapeDtypeStruct((N, Cz), jnp.float32),
      ],
  )(p, z, degp, Wr, b)


def _tc_final(p, z, degp, Wl):
  C = p.shape[2]
  n_out = Wl.shape[0]

  def body(p_ref, z_ref, d_ref, wl_ref, o_ref):
    pb = p_ref[...]
    deg = d_ref[0] + d_ref[1]
    dinv = 1.0 / jnp.maximum(deg, 1.0)
    agg = (pb[0] + pb[1]) * dinv
    dn = (((1,), (1,)), ((), ()))
    h = lax.dot_general(agg, wl_ref[...], dn,
                        preferred_element_type=jnp.float32) + z_ref[...]
    m = jnp.max(h, axis=-1, keepdims=True)
    lse = jnp.log(jnp.sum(jnp.exp(h - m), axis=-1, keepdims=True))
    o_ref[...] = h - m - lse

  grid = (N // ROWS_B,)
  return pl.pallas_call(
      body,
      grid=grid,
      in_specs=[
          pl.BlockSpec((NC, ROWS_B, C), lambda i: (0, i, 0)),
          pl.BlockSpec((ROWS_B, n_out), lambda i: (i, 0)),
          pl.BlockSpec((NC, ROWS_B, 1), lambda i: (0, i, 0)),
          pl.BlockSpec(Wl.shape, lambda i: (0, 0)),
      ],
      out_specs=pl.BlockSpec((ROWS_B, n_out), lambda i: (i, 0)),
      out_shape=jax.ShapeDtypeStruct((N, n_out), jnp.float32),
  )(p, z, degp, Wl)


@jax.jit
def kernel(x, adj_t, Wl0, Wr0, b0, Wl1, Wr1, b1, Wl2, Wr2, b2):
  src2d = adj_t[0].astype(jnp.int32).reshape(NW, NCHUNK, K)
  dst2d = adj_t[1].astype(jnp.int32).reshape(NW, NCHUNK, K)
  zrows = jnp.zeros((RPT_LAST, 128), jnp.float32)
  zdeg = jnp.zeros((N,), jnp.float32)

  y0, z0 = _tc_matmul2(x, Wl0, Wr0, b0.reshape(1, -1))
  p0, degp = _sc_segment_sum(y0, src2d, dst2d, zrows, zdeg, True)
  degp3 = degp.reshape(NC, N, 1)
  y1, z1 = _tc_combine_matmul2(p0, z0, degp3, Wl1, Wr1, b1.reshape(1, -1))
  (p1,) = _sc_segment_sum(y1, src2d, dst2d, zrows, zdeg, False)
  h2, z2 = _tc_combine_hz(p1, z1, degp3, Wr2, b2.reshape(1, -1))
  (p2,) = _sc_segment_sum(h2, src2d, dst2d, zrows, zdeg, False)
  return _tc_final(p2, z2, degp3, Wl2)


# speedup vs baseline: 9.4930x; 1.3061x over previous
import functools

import jax
import jax.numpy as jnp
from jax import lax
from jax.experimental import pallas as pl
from jax.experimental.pallas import tpu as pltpu
from jax.experimental.pallas import tpu_sc as plsc

N = 10000
E = 320000
NC = 2
NS = 16
NW = NC * NS
ET = E // NW
K = 80
NCHUNK = ET // K
RPT = 624
RPT_LAST = N - RPT * (NS - 1)


def _sc_segment_sum(y, src2d, dst2d, zrows, zdeg, with_deg):
  C = y.shape[1]
  mesh = plsc.VectorSubcoreMesh(core_axis_name="c", subcore_axis_name="s")
  out_type = [jax.ShapeDtypeStruct((NC, N, C), jnp.float32)]
  scratch = [
      pltpu.VMEM_SHARED((N, C), jnp.float32),
      pltpu.VMEM((K,), jnp.int32),
      pltpu.VMEM((K,), jnp.int32),
      pltpu.VMEM((K,), jnp.int32),
      pltpu.VMEM((K,), jnp.int32),
      pltpu.VMEM((K, C), jnp.float32),
      pltpu.VMEM((K, C), jnp.float32),
      pltpu.SemaphoreType.DMA,
      pltpu.SemaphoreType.DMA,
  ]
  if with_deg:
    out_type.append(jax.ShapeDtypeStruct((NC, N), jnp.float32))
    scratch += [
        pltpu.VMEM_SHARED((N,), jnp.float32),
        pltpu.VMEM((K,), jnp.float32),
    ]

  def body(y_h, s_h, d_h, zr_h, zd_h, *rest):
    if with_deg:
      (out_h, deg_h, acc, sA, sB, dA, dB, rows0, rows1, sem_g, sem_i,
       dacc, ones) = rest
    else:
      out_h, acc, sA, sB, dA, dB, rows0, rows1, sem_g, sem_i = rest
    cid = lax.axis_index("c")
    sid = lax.axis_index("s")
    wid = cid * NS + sid
    r0 = pl.multiple_of(sid * RPT, 8)
    @pl.when(sid < NS - 1)
    def _():
      pltpu.sync_copy(zr_h.at[pl.ds(0, RPT)], acc.at[pl.ds(r0, RPT)])
    @pl.when(sid == NS - 1)
    def _():
      pltpu.sync_copy(zr_h, acc.at[pl.ds(RPT * (NS - 1), RPT_LAST)])
    if with_deg:
      @pl.when(sid == 0)
      def _():
        pltpu.sync_copy(zd_h, dacc)
      for j in range(K // 16):
        ones[pl.ds(j * 16, 16)] = jnp.ones((16,), jnp.float32)
    plsc.subcore_barrier()

    ebase = wid * ET

    def idxload(j, sbuf, dbuf):
      off = pl.multiple_of(ebase + j * K, 8)
      pltpu.async_copy(s_h.at[pl.ds(off, K)], sbuf, sem_i)
      pltpu.async_copy(d_h.at[pl.ds(off, K)], dbuf, sem_i)

    def idxwait(sbuf, dbuf):
      pltpu.make_async_copy(s_h.at[pl.ds(0, K)], sbuf, sem_i).wait()
      pltpu.make_async_copy(d_h.at[pl.ds(0, K)], dbuf, sem_i).wait()

    def gather(sbuf, buf):
      pltpu.async_copy(y_h.at[sbuf], buf, sem_g)

    def drain(buf):
      pltpu.make_async_copy(y_h.at[sA], buf, sem_g).wait()

    def scat(dbuf, buf):
      pltpu.sync_copy(buf, acc.at[dbuf], add=True)
      if with_deg:
        pltpu.sync_copy(ones, dacc.at[dbuf], add=True)

    def half(j, sP, dP, rowsP, sQ, dQ, rowsQ, last):
      if not last:
        idxwait(sQ, dQ)
        drain(rowsP)
        gather(sQ, rowsQ)
        scat(dP, rowsP)
        @pl.when(j + 2 < NCHUNK)
        def _():
          idxload(j + 2, sP, dP)
      else:
        drain(rowsP)
        scat(dP, rowsP)

    idxload(0, sA, dA)
    idxwait(sA, dA)
    gather(sA, rows0)
    idxload(1, sB, dB)

    def step(i, carry):
      j = i * 2
      half(j, sA, dA, rows0, sB, dB, rows1, False)
      half(j + 1, sB, dB, rows1, sA, dA, rows0, False)
      return carry

    lax.fori_loop(0, (NCHUNK - 1) // 2, step, 0)
    half(NCHUNK - 1, sA, dA, rows0, sB, dB, rows1, True)
    plsc.subcore_barrier()
    @pl.when(sid < NS - 1)
    def _():
      pltpu.sync_copy(acc.at[pl.ds(r0, RPT)], out_h.at[cid, pl.ds(r0, RPT)])
    @pl.when(sid == NS - 1)
    def _():
      last = RPT * (NS - 1)
      pltpu.sync_copy(acc.at[pl.ds(last, RPT_LAST)],
                      out_h.at[cid, pl.ds(last, RPT_LAST)])
    if with_deg:
      @pl.when(sid == 0)
      def _():
        pltpu.sync_copy(dacc, deg_h.at[cid])

  fn = pl.kernel(body, out_type=out_type, mesh=mesh, scratch_types=scratch)
  return fn(y, src2d, dst2d, zrows, zdeg)


ROWS_B = 1000


def _tc_matmul2(x, Wl, Wr, b):
  Co = Wl.shape[0]
  Cz = Wr.shape[0]

  def body(x_ref, wl_ref, wr_ref, b_ref, y_ref, z_ref):
    xb = x_ref[...]
    dn = (((1,), (1,)), ((), ()))
    y_ref[...] = lax.dot_general(xb, wl_ref[...], dn,
                                 preferred_element_type=jnp.float32)
    z_ref[...] = lax.dot_general(xb, wr_ref[...], dn,
                                 preferred_element_type=jnp.float32) + b_ref[...]

  grid = (N // ROWS_B,)
  return pl.pallas_call(
      body,
      grid=grid,
      in_specs=[
          pl.BlockSpec((ROWS_B, x.shape[1]), lambda i: (i, 0)),
          pl.BlockSpec(Wl.shape, lambda i: (0, 0)),
          pl.BlockSpec(Wr.shape, lambda i: (0, 0)),
          pl.BlockSpec((1, Cz), lambda i: (0, 0)),
      ],
      out_specs=[
          pl.BlockSpec((ROWS_B, Co), lambda i: (i, 0)),
          pl.BlockSpec((ROWS_B, Cz), lambda i: (i, 0)),
      ],
      out_shape=[
          jax.ShapeDtypeStruct((N, Co), jnp.float32),
          jax.ShapeDtypeStruct((N, Cz), jnp.float32),
      ],
  )(x, Wl, Wr, b)


def _tc_combine_matmul2(p, z, degp, Wl, Wr, b):
  C = z.shape[1]
  Co = Wl.shape[0]
  Cz = Wr.shape[0]

  def body(p_ref, z_ref, d_ref, wl_ref, wr_ref, b_ref, y_ref, z2_ref):
    pb = p_ref[...]
    deg = d_ref[0] + d_ref[1]
    dinv = 1.0 / jnp.maximum(deg, 1.0)
    h = jnp.maximum((pb[0] + pb[1]) * dinv + z_ref[...], 0.0)
    dn = (((1,), (1,)), ((), ()))
    y_ref[...] = lax.dot_general(h, wl_ref[...], dn,
                                 preferred_element_type=jnp.float32)
    z2_ref[...] = lax.dot_general(h, wr_ref[...], dn,
                                  preferred_element_type=jnp.float32) + b_ref[...]

  grid = (N // ROWS_B,)
  return pl.pallas_call(
      body,
      grid=grid,
      in_specs=[
          pl.BlockSpec((NC, ROWS_B, C), lambda i: (0, i, 0)),
          pl.BlockSpec((ROWS_B, C), lambda i: (i, 0)),
          pl.BlockSpec((NC, ROWS_B, 1), lambda i: (0, i, 0)),
          pl.BlockSpec(Wl.shape, lambda i: (0, 0)),
          pl.BlockSpec(Wr.shape, lambda i: (0, 0)),
          pl.BlockSpec((1, Cz), lambda i: (0, 0)),
      ],
      out_specs=[
          pl.BlockSpec((ROWS_B, Co), lambda i: (i, 0)),
          pl.BlockSpec((ROWS_B, Cz), lambda i: (i, 0)),
      ],
      out_shape=[
          jax.ShapeDtypeStruct((N, Co), jnp.float32),
          jax.ShapeDtypeStruct((N, Cz), jnp.float32),
      ],
  )(p, z, degp, Wl, Wr, b)


def _tc_combine_hz(p, z, degp, Wr, b):
  C = z.shape[1]
  Cz = Wr.shape[0]

  def body(p_ref, z_ref, d_ref, wr_ref, b_ref, h_ref, z2_ref):
    pb = p_ref[...]
    deg = d_ref[0] + d_ref[1]
    dinv = 1.0 / jnp.maximum(deg, 1.0)
    h = jnp.maximum((pb[0] + pb[1]) * dinv + z_ref[...], 0.0)
    h_ref[...] = h
    dn = (((1,), (1,)), ((), ()))
    z2_ref[...] = lax.dot_general(h, wr_ref[...], dn,
                                  preferred_element_type=jnp.float32) + b_ref[...]

  grid = (N // ROWS_B,)
  return pl.pallas_call(
      body,
      grid=grid,
      in_specs=[
          pl.BlockSpec((NC, ROWS_B, C), lambda i: (0, i, 0)),
          pl.BlockSpec((ROWS_B, C), lambda i: (i, 0)),
          pl.BlockSpec((NC, ROWS_B, 1), lambda i: (0, i, 0)),
          pl.BlockSpec(Wr.shape, lambda i: (0, 0)),
          pl.BlockSpec((1, Cz), lambda i: (0, 0)),
      ],
      out_specs=[
          pl.BlockSpec((ROWS_B, C), lambda i: (i, 0)),
          pl.BlockSpec((ROWS_B, Cz), lambda i: (i, 0)),
      ],
      out_shape=[
          jax.ShapeDtypeStruct((N, C), jnp.float32),
          jax.ShapeDtypeStruct((N, Cz), jnp.float32),
      ],
  )(p, z, degp, Wr, b)


def _tc_final(p, z, degp, Wl):
  C = p.shape[2]
  n_out = Wl.shape[0]

  def body(p_ref, z_ref, d_ref, wl_ref, o_ref):
    pb = p_ref[...]
    deg = d_ref[0] + d_ref[1]
    dinv = 1.0 / jnp.maximum(deg, 1.0)
    agg = (pb[0] + pb[1]) * dinv
    dn = (((1,), (1,)), ((), ()))
    h = lax.dot_general(agg, wl_ref[...], dn,
                        preferred_element_type=jnp.float32) + z_ref[...]
    m = jnp.max(h, axis=-1, keepdims=True)
    lse = jnp.log(jnp.sum(jnp.exp(h - m), axis=-1, keepdims=True))
    o_ref[...] = h - m - lse

  grid = (N // ROWS_B,)
  return pl.pallas_call(
      body,
      grid=grid,
      in_specs=[
          pl.BlockSpec((NC, ROWS_B, C), lambda i: (0, i, 0)),
          pl.BlockSpec((ROWS_B, n_out), lambda i: (i, 0)),
          pl.BlockSpec((NC, ROWS_B, 1), lambda i: (0, i, 0)),
          pl.BlockSpec(Wl.shape, lambda i: (0, 0)),
      ],
      out_specs=pl.BlockSpec((ROWS_B, n_out), lambda i: (i, 0)),
      out_shape=jax.ShapeDtypeStruct((N, n_out), jnp.float32),
  )(p, z, degp, Wl)


@jax.jit
def kernel(x, adj_t, Wl0, Wr0, b0, Wl1, Wr1, b1, Wl2, Wr2, b2):
  src2d = adj_t[0].astype(jnp.int32)
  dst2d = adj_t[1].astype(jnp.int32)
  zrows = jnp.zeros((RPT_LAST, 128), jnp.float32)
  zdeg = jnp.zeros((N,), jnp.float32)

  y0, z0 = _tc_matmul2(x, Wl0, Wr0, b0.reshape(1, -1))
  p0, degp = _sc_segment_sum(y0, src2d, dst2d, zrows, zdeg, True)
  degp3 = degp.reshape(NC, N, 1)
  y1, z1 = _tc_combine_matmul2(p0, z0, degp3, Wl1, Wr1, b1.reshape(1, -1))
  (p1,) = _sc_segment_sum(y1, src2d, dst2d, zrows, zdeg, False)
  h2, z2 = _tc_combine_hz(p1, z1, degp3, Wr2, b2.reshape(1, -1))
  (p2,) = _sc_segment_sum(h2, src2d, dst2d, zrows, zdeg, False)
  return _tc_final(p2, z2, degp3, Wl2)


# speedup vs baseline: 11.2270x; 1.1827x over previous
import functools

import jax
import jax.numpy as jnp
from jax import lax
from jax.experimental import pallas as pl
from jax.experimental.pallas import tpu as pltpu
from jax.experimental.pallas import tpu_sc as plsc

N = 10000
E = 320000
NC = 2
NS = 16
NW = NC * NS
ET = E // NW
K = 128
NCHUNK = ET // K
KT = ET - NCHUNK * K
RPT = 624
RPT_LAST = N - RPT * (NS - 1)


def _sc_segment_sum(y, src2d, dst2d, zrows, zdeg, with_deg):
  C = y.shape[1]
  mesh = plsc.VectorSubcoreMesh(core_axis_name="c", subcore_axis_name="s")
  out_type = [jax.ShapeDtypeStruct((NC, N, C), jnp.float32)]
  scratch = [
      pltpu.VMEM_SHARED((N, C), jnp.float32),
      pltpu.VMEM((K,), jnp.int32),
      pltpu.VMEM((K,), jnp.int32),
      pltpu.VMEM((K,), jnp.int32),
      pltpu.VMEM((K,), jnp.int32),
      pltpu.VMEM((KT,), jnp.int32),
      pltpu.VMEM((KT,), jnp.int32),
      pltpu.VMEM((K, C), jnp.float32),
      pltpu.VMEM((K, C), jnp.float32),
      pltpu.VMEM((KT, C), jnp.float32),
      pltpu.SemaphoreType.DMA,
      pltpu.SemaphoreType.DMA,
  ]
  if with_deg:
    out_type.append(jax.ShapeDtypeStruct((NC, N), jnp.float32))
    scratch += [
        pltpu.VMEM_SHARED((N,), jnp.float32),
        pltpu.VMEM((K,), jnp.float32),
    ]

  def body(y_h, s_h, d_h, zr_h, zd_h, *rest):
    if with_deg:
      (out_h, deg_h, acc, sA, sB, dA, dB, sT, dT, rows0, rows1, rowsT,
       sem_g, sem_i, dacc, ones) = rest
    else:
      (out_h, acc, sA, sB, dA, dB, sT, dT, rows0, rows1, rowsT,
       sem_g, sem_i) = rest
    cid = lax.axis_index("c")
    sid = lax.axis_index("s")
    wid = cid * NS + sid
    r0 = pl.multiple_of(sid * RPT, 8)
    @pl.when(sid < NS - 1)
    def _():
      pltpu.sync_copy(zr_h.at[pl.ds(0, RPT)], acc.at[pl.ds(r0, RPT)])
    @pl.when(sid == NS - 1)
    def _():
      pltpu.sync_copy(zr_h, acc.at[pl.ds(RPT * (NS - 1), RPT_LAST)])
    if with_deg:
      @pl.when(sid == 0)
      def _():
        pltpu.sync_copy(zd_h, dacc)
      for j in range(K // 16):
        ones[pl.ds(j * 16, 16)] = jnp.ones((16,), jnp.float32)
    plsc.subcore_barrier()

    ebase = wid * ET

    def idxload(j, sbuf, dbuf):
      off = pl.multiple_of(ebase + j * K, 8)
      pltpu.async_copy(s_h.at[pl.ds(off, K)], sbuf, sem_i)
      pltpu.async_copy(d_h.at[pl.ds(off, K)], dbuf, sem_i)

    def idxwait(sbuf, dbuf):
      pltpu.make_async_copy(s_h.at[pl.ds(0, K)], sbuf, sem_i).wait()
      pltpu.make_async_copy(d_h.at[pl.ds(0, K)], dbuf, sem_i).wait()

    def gather(sbuf, buf):
      pltpu.async_copy(y_h.at[sbuf], buf, sem_g)

    def drain(buf):
      pltpu.make_async_copy(y_h.at[sA], buf, sem_g).wait()

    def scat(dbuf, buf):
      pltpu.sync_copy(buf, acc.at[dbuf], add=True)
      if with_deg:
        pltpu.sync_copy(ones, dacc.at[dbuf], add=True)

    def half(j, sP, dP, rowsP, sQ, dQ, rowsQ, last):
      if not last:
        idxwait(sQ, dQ)
        drain(rowsP)
        gather(sQ, rowsQ)
        scat(dP, rowsP)
        @pl.when(j + 2 < NCHUNK)
        def _():
          idxload(j + 2, sP, dP)
      else:
        drain(rowsP)
        scat(dP, rowsP)

    idxload(0, sA, dA)
    idxwait(sA, dA)
    gather(sA, rows0)
    idxload(1, sB, dB)

    def step(i, carry):
      j = i * 2
      half(j, sA, dA, rows0, sB, dB, rows1, False)
      half(j + 1, sB, dB, rows1, sA, dA, rows0, False)
      return carry

    lax.fori_loop(0, (NCHUNK - 2) // 2, step, 0)
    half(NCHUNK - 2, sA, dA, rows0, sB, dB, rows1, False)
    half(NCHUNK - 1, sB, dB, rows1, sA, dA, rows0, True)
    toff = pl.multiple_of(ebase + NCHUNK * K, 8)
    pltpu.async_copy(s_h.at[pl.ds(toff, KT)], sT, sem_i)
    pltpu.async_copy(d_h.at[pl.ds(toff, KT)], dT, sem_i)
    pltpu.make_async_copy(s_h.at[pl.ds(0, KT)], sT, sem_i).wait()
    pltpu.make_async_copy(d_h.at[pl.ds(0, KT)], dT, sem_i).wait()
    pltpu.async_copy(y_h.at[sT], rowsT, sem_g)
    pltpu.make_async_copy(y_h.at[sT], rowsT, sem_g).wait()
    pltpu.sync_copy(rowsT, acc.at[dT], add=True)
    if with_deg:
      pltpu.sync_copy(ones.at[pl.ds(0, KT)], dacc.at[dT], add=True)
    plsc.subcore_barrier()
    @pl.when(sid < NS - 1)
    def _():
      pltpu.sync_copy(acc.at[pl.ds(r0, RPT)], out_h.at[cid, pl.ds(r0, RPT)])
    @pl.when(sid == NS - 1)
    def _():
      last = RPT * (NS - 1)
      pltpu.sync_copy(acc.at[pl.ds(last, RPT_LAST)],
                      out_h.at[cid, pl.ds(last, RPT_LAST)])
    if with_deg:
      @pl.when(sid == 0)
      def _():
        pltpu.sync_copy(dacc, deg_h.at[cid])

  fn = pl.kernel(body, out_type=out_type, mesh=mesh, scratch_types=scratch)
  return fn(y, src2d, dst2d, zrows, zdeg)


ROWS_B = 2000


def _tc_matmul2(x, Wl, Wr, b):
  Co = Wl.shape[0]
  Cz = Wr.shape[0]

  def body(x_ref, wl_ref, wr_ref, b_ref, y_ref, z_ref):
    xb = x_ref[...]
    dn = (((1,), (1,)), ((), ()))
    y_ref[...] = lax.dot_general(xb, wl_ref[...], dn,
                                 preferred_element_type=jnp.float32)
    z_ref[...] = lax.dot_general(xb, wr_ref[...], dn,
                                 preferred_element_type=jnp.float32) + b_ref[...]

  grid = (N // ROWS_B,)
  return pl.pallas_call(
      body,
      grid=grid,
      in_specs=[
          pl.BlockSpec((ROWS_B, x.shape[1]), lambda i: (i, 0)),
          pl.BlockSpec(Wl.shape, lambda i: (0, 0)),
          pl.BlockSpec(Wr.shape, lambda i: (0, 0)),
          pl.BlockSpec((1, Cz), lambda i: (0, 0)),
      ],
      out_specs=[
          pl.BlockSpec((ROWS_B, Co), lambda i: (i, 0)),
          pl.BlockSpec((ROWS_B, Cz), lambda i: (i, 0)),
      ],
      out_shape=[
          jax.ShapeDtypeStruct((N, Co), jnp.float32),
          jax.ShapeDtypeStruct((N, Cz), jnp.float32),
      ],
  )(x, Wl, Wr, b)


def _tc_combine_matmul2(p, z, degp, Wl, Wr, b):
  C = z.shape[1]
  Co = Wl.shape[0]
  Cz = Wr.shape[0]

  def body(p_ref, z_ref, d_ref, wl_ref, wr_ref, b_ref, y_ref, z2_ref):
    pb = p_ref[...]
    deg = d_ref[0] + d_ref[1]
    dinv = 1.0 / jnp.maximum(deg, 1.0)
    h = jnp.maximum((pb[0] + pb[1]) * dinv + z_ref[...], 0.0)
    dn = (((1,), (1,)), ((), ()))
    y_ref[...] = lax.dot_general(h, wl_ref[...], dn,
                                 preferred_element_type=jnp.float32)
    z2_ref[...] = lax.dot_general(h, wr_ref[...], dn,
                                  preferred_element_type=jnp.float32) + b_ref[...]

  grid = (N // ROWS_B,)
  return pl.pallas_call(
      body,
      grid=grid,
      in_specs=[
          pl.BlockSpec((NC, ROWS_B, C), lambda i: (0, i, 0)),
          pl.BlockSpec((ROWS_B, C), lambda i: (i, 0)),
          pl.BlockSpec((NC, ROWS_B, 1), lambda i: (0, i, 0)),
          pl.BlockSpec(Wl.shape, lambda i: (0, 0)),
          pl.BlockSpec(Wr.shape, lambda i: (0, 0)),
          pl.BlockSpec((1, Cz), lambda i: (0, 0)),
      ],
      out_specs=[
          pl.BlockSpec((ROWS_B, Co), lambda i: (i, 0)),
          pl.BlockSpec((ROWS_B, Cz), lambda i: (i, 0)),
      ],
      out_shape=[
          jax.ShapeDtypeStruct((N, Co), jnp.float32),
          jax.ShapeDtypeStruct((N, Cz), jnp.float32),
      ],
  )(p, z, degp, Wl, Wr, b)


def _tc_combine_hz(p, z, degp, Wr, b):
  C = z.shape[1]
  Cz = Wr.shape[0]

  def body(p_ref, z_ref, d_ref, wr_ref, b_ref, h_ref, z2_ref):
    pb = p_ref[...]
    deg = d_ref[0] + d_ref[1]
    dinv = 1.0 / jnp.maximum(deg, 1.0)
    h = jnp.maximum((pb[0] + pb[1]) * dinv + z_ref[...], 0.0)
    h_ref[...] = h
    dn = (((1,), (1,)), ((), ()))
    z2_ref[...] = lax.dot_general(h, wr_ref[...], dn,
                                  preferred_element_type=jnp.float32) + b_ref[...]

  grid = (N // ROWS_B,)
  return pl.pallas_call(
      body,
      grid=grid,
      in_specs=[
          pl.BlockSpec((NC, ROWS_B, C), lambda i: (0, i, 0)),
          pl.BlockSpec((ROWS_B, C), lambda i: (i, 0)),
          pl.BlockSpec((NC, ROWS_B, 1), lambda i: (0, i, 0)),
          pl.BlockSpec(Wr.shape, lambda i: (0, 0)),
          pl.BlockSpec((1, Cz), lambda i: (0, 0)),
      ],
      out_specs=[
          pl.BlockSpec((ROWS_B, C), lambda i: (i, 0)),
          pl.BlockSpec((ROWS_B, Cz), lambda i: (i, 0)),
      ],
      out_shape=[
          jax.ShapeDtypeStruct((N, C), jnp.float32),
          jax.ShapeDtypeStruct((N, Cz), jnp.float32),
      ],
  )(p, z, degp, Wr, b)


def _tc_final(p, z, degp, Wl):
  C = p.shape[2]
  n_out = Wl.shape[0]

  def body(p_ref, z_ref, d_ref, wl_ref, o_ref):
    pb = p_ref[...]
    deg = d_ref[0] + d_ref[1]
    dinv = 1.0 / jnp.maximum(deg, 1.0)
    agg = (pb[0] + pb[1]) * dinv
    dn = (((1,), (1,)), ((), ()))
    h = lax.dot_general(agg, wl_ref[...], dn,
                        preferred_element_type=jnp.float32) + z_ref[...]
    m = jnp.max(h, axis=-1, keepdims=True)
    lse = jnp.log(jnp.sum(jnp.exp(h - m), axis=-1, keepdims=True))
    o_ref[...] = h - m - lse

  grid = (N // ROWS_B,)
  return pl.pallas_call(
      body,
      grid=grid,
      in_specs=[
          pl.BlockSpec((NC, ROWS_B, C), lambda i: (0, i, 0)),
          pl.BlockSpec((ROWS_B, n_out), lambda i: (i, 0)),
          pl.BlockSpec((NC, ROWS_B, 1), lambda i: (0, i, 0)),
          pl.BlockSpec(Wl.shape, lambda i: (0, 0)),
      ],
      out_specs=pl.BlockSpec((ROWS_B, n_out), lambda i: (i, 0)),
      out_shape=jax.ShapeDtypeStruct((N, n_out), jnp.float32),
  )(p, z, degp, Wl)


@jax.jit
def kernel(x, adj_t, Wl0, Wr0, b0, Wl1, Wr1, b1, Wl2, Wr2, b2):
  src2d = adj_t[0].astype(jnp.int32)
  dst2d = adj_t[1].astype(jnp.int32)
  zrows = jnp.zeros((RPT_LAST, 128), jnp.float32)
  zdeg = jnp.zeros((N,), jnp.float32)

  y0, z0 = _tc_matmul2(x, Wl0, Wr0, b0.reshape(1, -1))
  p0, degp = _sc_segment_sum(y0, src2d, dst2d, zrows, zdeg, True)
  degp3 = degp.reshape(NC, N, 1)
  y1, z1 = _tc_combine_matmul2(p0, z0, degp3, Wl1, Wr1, b1.reshape(1, -1))
  (p1,) = _sc_segment_sum(y1, src2d, dst2d, zrows, zdeg, False)
  h2, z2 = _tc_combine_hz(p1, z1, degp3, Wr2, b2.reshape(1, -1))
  (p2,) = _sc_segment_sum(h2, src2d, dst2d, zrows, zdeg, False)
  return _tc_final(p2, z2, degp3, Wl2)


# speedup vs baseline: 11.2772x; 1.0045x over previous
import functools

import jax
import jax.numpy as jnp
from jax import lax
from jax.experimental import pallas as pl
from jax.experimental.pallas import tpu as pltpu
from jax.experimental.pallas import tpu_sc as plsc

N = 10000
E = 320000
NC = 2
NS = 16
NW = NC * NS
ET = E // NW
K = 128
NCHUNK = ET // K
KT = ET - NCHUNK * K
RPT = 624
RPT_LAST = N - RPT * (NS - 1)


def _sc_segment_sum(y, src2d, dst2d, zrows, zdeg, with_deg):
  C = y.shape[1]
  mesh = plsc.VectorSubcoreMesh(core_axis_name="c", subcore_axis_name="s")
  out_type = [jax.ShapeDtypeStruct((NC, N, C), jnp.float32)]
  scratch = [
      pltpu.VMEM_SHARED((N, C), jnp.float32),
      pltpu.VMEM((K,), jnp.int32),
      pltpu.VMEM((K,), jnp.int32),
      pltpu.VMEM((K,), jnp.int32),
      pltpu.VMEM((K,), jnp.int32),
      pltpu.VMEM((K,), jnp.int32),
      pltpu.VMEM((K,), jnp.int32),
      pltpu.VMEM((KT,), jnp.int32),
      pltpu.VMEM((KT,), jnp.int32),
      pltpu.VMEM((K, C), jnp.float32),
      pltpu.VMEM((K, C), jnp.float32),
      pltpu.VMEM((KT, C), jnp.float32),
      pltpu.SemaphoreType.DMA,
      pltpu.SemaphoreType.DMA,
      pltpu.SemaphoreType.DMA,
  ]
  if with_deg:
    out_type.append(jax.ShapeDtypeStruct((NC, N), jnp.float32))
    scratch += [
        pltpu.VMEM_SHARED((N,), jnp.float32),
        pltpu.VMEM((K,), jnp.float32),
    ]

  def body(y_h, s_h, d_h, zr_h, zd_h, *rest):
    if with_deg:
      (out_h, deg_h, acc, sA, sB, d0, d1, d2, d3, sT, dT, rows0, rows1,
       rowsT, sem_g, sem_i, sem_s, dacc, ones) = rest
    else:
      (out_h, acc, sA, sB, d0, d1, d2, d3, sT, dT, rows0, rows1, rowsT,
       sem_g, sem_i, sem_s) = rest
    sbufs = (sA, sB)
    dbufs = (d0, d1, d2, d3)
    rbufs = (rows0, rows1)
    cid = lax.axis_index("c")
    sid = lax.axis_index("s")
    wid = cid * NS + sid
    r0 = pl.multiple_of(sid * RPT, 8)
    @pl.when(sid < NS - 1)
    def _():
      pltpu.sync_copy(zr_h.at[pl.ds(0, RPT)], acc.at[pl.ds(r0, RPT)])
    @pl.when(sid == NS - 1)
    def _():
      pltpu.sync_copy(zr_h, acc.at[pl.ds(RPT * (NS - 1), RPT_LAST)])
    if with_deg:
      @pl.when(sid == 0)
      def _():
        pltpu.sync_copy(zd_h, dacc)
      for j in range(K // 16):
        ones[pl.ds(j * 16, 16)] = jnp.ones((16,), jnp.float32)
    plsc.subcore_barrier()

    ebase = wid * ET

    def idxload(j, sbuf, dbuf):
      off = pl.multiple_of(ebase + j * K, 8)
      pltpu.async_copy(s_h.at[pl.ds(off, K)], sbuf, sem_i)
      pltpu.async_copy(d_h.at[pl.ds(off, K)], dbuf, sem_i)

    def idxwait():
      pltpu.make_async_copy(s_h.at[pl.ds(0, K)], sA, sem_i).wait()
      pltpu.make_async_copy(d_h.at[pl.ds(0, K)], d0, sem_i).wait()

    def gather(sbuf, buf):
      pltpu.async_copy(y_h.at[sbuf], buf, sem_g)

    def drain_g(buf):
      pltpu.make_async_copy(y_h.at[sA], buf, sem_g).wait()

    def scat(dbuf, buf):
      pltpu.async_copy(buf, acc.at[dbuf], sem_s, add=True)
      if with_deg:
        pltpu.sync_copy(ones, dacc.at[dbuf], add=True)

    def drain_s(dbuf, buf):
      pltpu.make_async_copy(buf, acc.at[dbuf], sem_s).wait()

    def half(j, u, first):
      idxwait()
      drain_g(rbufs[u % 2])
      if not first:
        drain_s(dbufs[(u + 3) % 4], rbufs[(u + 1) % 2])
      gather(sbufs[(u + 1) % 2], rbufs[(u + 1) % 2])
      scat(dbufs[u % 4], rbufs[u % 2])
      @pl.when(j + 2 < NCHUNK)
      def _():
        idxload(j + 2, sbufs[u % 2], dbufs[(u + 2) % 4])

    idxload(0, sA, d0)
    idxwait()
    gather(sA, rows0)
    idxload(1, sB, d1)
    half(0, 0, True)

    def step(i, carry):
      j = 1 + i * 4
      half(j, 1, False)
      half(j + 1, 2, False)
      half(j + 2, 3, False)
      half(j + 3, 0, False)
      return carry

    lax.fori_loop(0, (NCHUNK - 2) // 4, step, 0)
    drain_g(rows1)
    drain_s(dbufs[0], rows0)
    scat(dbufs[1], rows1)
    toff = pl.multiple_of(ebase + NCHUNK * K, 8)
    pltpu.async_copy(s_h.at[pl.ds(toff, KT)], sT, sem_i)
    pltpu.async_copy(d_h.at[pl.ds(toff, KT)], dT, sem_i)
    pltpu.make_async_copy(s_h.at[pl.ds(0, KT)], sT, sem_i).wait()
    pltpu.make_async_copy(d_h.at[pl.ds(0, KT)], dT, sem_i).wait()
    pltpu.async_copy(y_h.at[sT], rowsT, sem_g)
    pltpu.make_async_copy(y_h.at[sT], rowsT, sem_g).wait()
    pltpu.sync_copy(rowsT, acc.at[dT], add=True)
    if with_deg:
      pltpu.sync_copy(ones.at[pl.ds(0, KT)], dacc.at[dT], add=True)
    drain_s(dbufs[1], rows1)
    plsc.subcore_barrier()
    @pl.when(sid < NS - 1)
    def _():
      pltpu.sync_copy(acc.at[pl.ds(r0, RPT)], out_h.at[cid, pl.ds(r0, RPT)])
    @pl.when(sid == NS - 1)
    def _():
      last = RPT * (NS - 1)
      pltpu.sync_copy(acc.at[pl.ds(last, RPT_LAST)],
                      out_h.at[cid, pl.ds(last, RPT_LAST)])
    if with_deg:
      @pl.when(sid == 0)
      def _():
        pltpu.sync_copy(dacc, deg_h.at[cid])

  fn = pl.kernel(body, out_type=out_type, mesh=mesh, scratch_types=scratch)
  return fn(y, src2d, dst2d, zrows, zdeg)


ROWS_B = 2000


def _tc_matmul2(x, Wl, Wr, b):
  Co = Wl.shape[0]
  Cz = Wr.shape[0]

  def body(x_ref, wl_ref, wr_ref, b_ref, y_ref, z_ref):
    xb = x_ref[...]
    dn = (((1,), (1,)), ((), ()))
    y_ref[...] = lax.dot_general(xb, wl_ref[...], dn,
                                 preferred_element_type=jnp.float32)
    z_ref[...] = lax.dot_general(xb, wr_ref[...], dn,
                                 preferred_element_type=jnp.float32) + b_ref[...]

  grid = (N // ROWS_B,)
  return pl.pallas_call(
      body,
      grid=grid,
      in_specs=[
          pl.BlockSpec((ROWS_B, x.shape[1]), lambda i: (i, 0)),
          pl.BlockSpec(Wl.shape, lambda i: (0, 0)),
          pl.BlockSpec(Wr.shape, lambda i: (0, 0)),
          pl.BlockSpec((1, Cz), lambda i: (0, 0)),
      ],
      out_specs=[
          pl.BlockSpec((ROWS_B, Co), lambda i: (i, 0)),
          pl.BlockSpec((ROWS_B, Cz), lambda i: (i, 0)),
      ],
      out_shape=[
          jax.ShapeDtypeStruct((N, Co), jnp.float32),
          jax.ShapeDtypeStruct((N, Cz), jnp.float32),
      ],
  )(x, Wl, Wr, b)


def _tc_combine_matmul2(p, z, degp, Wl, Wr, b):
  C = z.shape[1]
  Co = Wl.shape[0]
  Cz = Wr.shape[0]

  def body(p_ref, z_ref, d_ref, wl_ref, wr_ref, b_ref, y_ref, z2_ref):
    pb = p_ref[...]
    deg = d_ref[0] + d_ref[1]
    dinv = 1.0 / jnp.maximum(deg, 1.0)
    h = jnp.maximum((pb[0] + pb[1]) * dinv + z_ref[...], 0.0)
    dn = (((1,), (1,)), ((), ()))
    y_ref[...] = lax.dot_general(h, wl_ref[...], dn,
                                 preferred_element_type=jnp.float32)
    z2_ref[...] = lax.dot_general(h, wr_ref[...], dn,
                                  preferred_element_type=jnp.float32) + b_ref[...]

  grid = (N // ROWS_B,)
  return pl.pallas_call(
      body,
      grid=grid,
      in_specs=[
          pl.BlockSpec((NC, ROWS_B, C), lambda i: (0, i, 0)),
          pl.BlockSpec((ROWS_B, C), lambda i: (i, 0)),
          pl.BlockSpec((NC, ROWS_B, 1), lambda i: (0, i, 0)),
          pl.BlockSpec(Wl.shape, lambda i: (0, 0)),
          pl.BlockSpec(Wr.shape, lambda i: (0, 0)),
          pl.BlockSpec((1, Cz), lambda i: (0, 0)),
      ],
      out_specs=[
          pl.BlockSpec((ROWS_B, Co), lambda i: (i, 0)),
          pl.BlockSpec((ROWS_B, Cz), lambda i: (i, 0)),
      ],
      out_shape=[
          jax.ShapeDtypeStruct((N, Co), jnp.float32),
          jax.ShapeDtypeStruct((N, Cz), jnp.float32),
      ],
  )(p, z, degp, Wl, Wr, b)


def _tc_combine_hz(p, z, degp, Wr, b):
  C = z.shape[1]
  Cz = Wr.shape[0]

  def body(p_ref, z_ref, d_ref, wr_ref, b_ref, h_ref, z2_ref):
    pb = p_ref[...]
    deg = d_ref[0] + d_ref[1]
    dinv = 1.0 / jnp.maximum(deg, 1.0)
    h = jnp.maximum((pb[0] + pb[1]) * dinv + z_ref[...], 0.0)
    h_ref[...] = h
    dn = (((1,), (1,)), ((), ()))
    z2_ref[...] = lax.dot_general(h, wr_ref[...], dn,
                                  preferred_element_type=jnp.float32) + b_ref[...]

  grid = (N // ROWS_B,)
  return pl.pallas_call(
      body,
      grid=grid,
      in_specs=[
          pl.BlockSpec((NC, ROWS_B, C), lambda i: (0, i, 0)),
          pl.BlockSpec((ROWS_B, C), lambda i: (i, 0)),
          pl.BlockSpec((NC, ROWS_B, 1), lambda i: (0, i, 0)),
          pl.BlockSpec(Wr.shape, lambda i: (0, 0)),
          pl.BlockSpec((1, Cz), lambda i: (0, 0)),
      ],
      out_specs=[
          pl.BlockSpec((ROWS_B, C), lambda i: (i, 0)),
          pl.BlockSpec((ROWS_B, Cz), lambda i: (i, 0)),
      ],
      out_shape=[
          jax.ShapeDtypeStruct((N, C), jnp.float32),
          jax.ShapeDtypeStruct((N, Cz), jnp.float32),
      ],
  )(p, z, degp, Wr, b)


def _tc_final(p, z, degp, Wl):
  C = p.shape[2]
  n_out = Wl.shape[0]

  def body(p_ref, z_ref, d_ref, wl_ref, o_ref):
    pb = p_ref[...]
    deg = d_ref[0] + d_ref[1]
    dinv = 1.0 / jnp.maximum(deg, 1.0)
    agg = (pb[0] + pb[1]) * dinv
    dn = (((1,), (1,)), ((), ()))
    h = lax.dot_general(agg, wl_ref[...], dn,
                        preferred_element_type=jnp.float32) + z_ref[...]
    m = jnp.max(h, axis=-1, keepdims=True)
    lse = jnp.log(jnp.sum(jnp.exp(h - m), axis=-1, keepdims=True))
    o_ref[...] = h - m - lse

  grid = (N // ROWS_B,)
  return pl.pallas_call(
      body,
      grid=grid,
      in_specs=[
          pl.BlockSpec((NC, ROWS_B, C), lambda i: (0, i, 0)),
          pl.BlockSpec((ROWS_B, n_out), lambda i: (i, 0)),
          pl.BlockSpec((NC, ROWS_B, 1), lambda i: (0, i, 0)),
          pl.BlockSpec(Wl.shape, lambda i: (0, 0)),
      ],
      out_specs=pl.BlockSpec((ROWS_B, n_out), lambda i: (i, 0)),
      out_shape=jax.ShapeDtypeStruct((N, n_out), jnp.float32),
  )(p, z, degp, Wl)


@jax.jit
def kernel(x, adj_t, Wl0, Wr0, b0, Wl1, Wr1, b1, Wl2, Wr2, b2):
  src2d = adj_t[0].astype(jnp.int32)
  dst2d = adj_t[1].astype(jnp.int32)
  zrows = jnp.zeros((RPT_LAST, 128), jnp.float32)
  zdeg = jnp.zeros((N,), jnp.float32)

  y0, z0 = _tc_matmul2(x, Wl0, Wr0, b0.reshape(1, -1))
  p0, degp = _sc_segment_sum(y0, src2d, dst2d, zrows, zdeg, True)
  degp3 = degp.reshape(NC, N, 1)
  y1, z1 = _tc_combine_matmul2(p0, z0, degp3, Wl1, Wr1, b1.reshape(1, -1))
  (p1,) = _sc_segment_sum(y1, src2d, dst2d, zrows, zdeg, False)
  h2, z2 = _tc_combine_hz(p1, z1, degp3, Wr2, b2.reshape(1, -1))
  (p2,) = _sc_segment_sum(h2, src2d, dst2d, zrows, zdeg, False)
  return _tc_final(p2, z2, degp3, Wl2)


# speedup vs baseline: 13.6821x; 1.2132x over previous
import functools

import jax
import jax.numpy as jnp
from jax import lax
from jax.experimental import pallas as pl
from jax.experimental.pallas import tpu as pltpu
from jax.experimental.pallas import tpu_sc as plsc

N = 10000
E = 320000
NC = 2
NS = 16
NW = NC * NS
ET = E // NW
K = 80
NCHUNK = ET // K
RPT = 624
RPT_LAST = N - RPT * (NS - 1)


def _sc_segment_sum(y, src2d, dst2d, zrows, zdeg, with_deg):
  C = y.shape[1]
  mesh = plsc.VectorSubcoreMesh(core_axis_name="c", subcore_axis_name="s")
  out_type = [jax.ShapeDtypeStruct((NC, N, C), jnp.float32)]
  scratch = [
      pltpu.VMEM_SHARED((N, C), jnp.float32),
      pltpu.VMEM((K,), jnp.int32),
      pltpu.VMEM((K,), jnp.int32),
      pltpu.VMEM((K,), jnp.int32),
      pltpu.VMEM((K,), jnp.int32),
      pltpu.VMEM((K,), jnp.int32),
      pltpu.VMEM((K,), jnp.int32),
      pltpu.VMEM((K,), jnp.int32),
      pltpu.VMEM((K,), jnp.int32),
      pltpu.VMEM((K, C), jnp.float32),
      pltpu.VMEM((K, C), jnp.float32),
      pltpu.VMEM((K, C), jnp.float32),
      pltpu.VMEM((K, C), jnp.float32),
      pltpu.SemaphoreType.DMA,
      pltpu.SemaphoreType.DMA,
      pltpu.SemaphoreType.DMA,
  ]
  if with_deg:
    out_type.append(jax.ShapeDtypeStruct((NC, N), jnp.float32))
    scratch += [
        pltpu.VMEM_SHARED((N,), jnp.float32),
        pltpu.VMEM((K,), jnp.float32),
    ]

  def body(y_h, s_h, d_h, zr_h, zd_h, *rest):
    if with_deg:
      (out_h, deg_h, acc, s0, s1, s2, s3, d0, d1, d2, d3, r0_, r1_, r2_, r3_,
       sem_g, sem_i, sem_s, dacc, ones) = rest
    else:
      (out_h, acc, s0, s1, s2, s3, d0, d1, d2, d3, r0_, r1_, r2_, r3_,
       sem_g, sem_i, sem_s) = rest
    sbufs = (s0, s1, s2, s3)
    dbufs = (d0, d1, d2, d3)
    rbufs = (r0_, r1_, r2_, r3_)
    cid = lax.axis_index("c")
    sid = lax.axis_index("s")
    wid = cid * NS + sid
    r0 = pl.multiple_of(sid * RPT, 8)
    @pl.when(sid < NS - 1)
    def _():
      pltpu.sync_copy(zr_h.at[pl.ds(0, RPT)], acc.at[pl.ds(r0, RPT)])
    @pl.when(sid == NS - 1)
    def _():
      pltpu.sync_copy(zr_h, acc.at[pl.ds(RPT * (NS - 1), RPT_LAST)])
    if with_deg:
      @pl.when(sid == 0)
      def _():
        pltpu.sync_copy(zd_h, dacc)
      for j in range(K // 16):
        ones[pl.ds(j * 16, 16)] = jnp.ones((16,), jnp.float32)
    plsc.subcore_barrier()

    ebase = wid * ET

    def idxload(j, u):
      off = pl.multiple_of(ebase + j * K, 8)
      pltpu.async_copy(s_h.at[pl.ds(off, K)], sbufs[u], sem_i)
      pltpu.async_copy(d_h.at[pl.ds(off, K)], dbufs[u], sem_i)

    def idxwait():
      pltpu.make_async_copy(s_h.at[pl.ds(0, K)], s0, sem_i).wait()
      pltpu.make_async_copy(d_h.at[pl.ds(0, K)], d0, sem_i).wait()

    def gather(u):
      pltpu.async_copy(y_h.at[sbufs[u]], rbufs[u], sem_g)

    def drain_g(u):
      pltpu.make_async_copy(y_h.at[s0], rbufs[u], sem_g).wait()

    def scat(u):
      pltpu.async_copy(rbufs[u], acc.at[dbufs[u]], sem_s, add=True)
      if with_deg:
        pltpu.sync_copy(ones, dacc.at[dbufs[u]], add=True)

    def drain_s(u):
      pltpu.make_async_copy(rbufs[u], acc.at[dbufs[u]], sem_s).wait()

    def half(j, u, first=False, wait_idx=True, do_gather=True, load=True):
      drain_g(u)
      if wait_idx:
        idxwait()
      if not first:
        drain_s((u + 3) % 4)
      if do_gather:
        gather((u + 2) % 4)
      scat(u)
      if load:
        idxload(j + 3, (u + 3) % 4)

    idxload(0, 0)
    idxwait()
    gather(0)
    idxload(1, 1)
    idxwait()
    gather(1)
    idxload(2, 2)
    half(0, 0, first=True)

    def step(i, carry):
      j = 1 + i * 4
      half(j, 1)
      half(j + 1, 2)
      half(j + 2, 3)
      half(j + 3, 0)
      return carry

    lax.fori_loop(0, (NCHUNK - 5) // 4, step, 0)
    half(NCHUNK - 4, 1)
    half(NCHUNK - 3, 2, load=False)
    half(NCHUNK - 2, 3, wait_idx=False, do_gather=False, load=False)
    half(NCHUNK - 1, 0, wait_idx=False, do_gather=False, load=False)
    drain_s(0)
    plsc.subcore_barrier()
    @pl.when(sid < NS - 1)
    def _():
      pltpu.sync_copy(acc.at[pl.ds(r0, RPT)], out_h.at[cid, pl.ds(r0, RPT)])
    @pl.when(sid == NS - 1)
    def _():
      last = RPT * (NS - 1)
      pltpu.sync_copy(acc.at[pl.ds(last, RPT_LAST)],
                      out_h.at[cid, pl.ds(last, RPT_LAST)])
    if with_deg:
      @pl.when(sid == 0)
      def _():
        pltpu.sync_copy(dacc, deg_h.at[cid])

  fn = pl.kernel(body, out_type=out_type, mesh=mesh, scratch_types=scratch)
  return fn(y, src2d, dst2d, zrows, zdeg)


ROWS_B = 2000


def _tc_matmul2(x, Wl, Wr, b):
  Co = Wl.shape[0]
  Cz = Wr.shape[0]

  def body(x_ref, wl_ref, wr_ref, b_ref, y_ref, z_ref):
    xb = x_ref[...]
    dn = (((1,), (1,)), ((), ()))
    y_ref[...] = lax.dot_general(xb, wl_ref[...], dn,
                                 preferred_element_type=jnp.float32)
    z_ref[...] = lax.dot_general(xb, wr_ref[...], dn,
                                 preferred_element_type=jnp.float32) + b_ref[...]

  grid = (N // ROWS_B,)
  return pl.pallas_call(
      body,
      grid=grid,
      in_specs=[
          pl.BlockSpec((ROWS_B, x.shape[1]), lambda i: (i, 0)),
          pl.BlockSpec(Wl.shape, lambda i: (0, 0)),
          pl.BlockSpec(Wr.shape, lambda i: (0, 0)),
          pl.BlockSpec((1, Cz), lambda i: (0, 0)),
      ],
      out_specs=[
          pl.BlockSpec((ROWS_B, Co), lambda i: (i, 0)),
          pl.BlockSpec((ROWS_B, Cz), lambda i: (i, 0)),
      ],
      out_shape=[
          jax.ShapeDtypeStruct((N, Co), jnp.float32),
          jax.ShapeDtypeStruct((N, Cz), jnp.float32),
      ],
  )(x, Wl, Wr, b)


def _tc_combine_matmul2(p, z, degp, Wl, Wr, b):
  C = z.shape[1]
  Co = Wl.shape[0]
  Cz = Wr.shape[0]

  def body(p_ref, z_ref, d_ref, wl_ref, wr_ref, b_ref, y_ref, z2_ref):
    pb = p_ref[...]
    deg = d_ref[0] + d_ref[1]
    dinv = 1.0 / jnp.maximum(deg, 1.0)
    h = jnp.maximum((pb[0] + pb[1]) * dinv + z_ref[...], 0.0)
    dn = (((1,), (1,)), ((), ()))
    y_ref[...] = lax.dot_general(h, wl_ref[...], dn,
                                 preferred_element_type=jnp.float32)
    z2_ref[...] = lax.dot_general(h, wr_ref[...], dn,
                                  preferred_element_type=jnp.float32) + b_ref[...]

  grid = (N // ROWS_B,)
  return pl.pallas_call(
      body,
      grid=grid,
      in_specs=[
          pl.BlockSpec((NC, ROWS_B, C), lambda i: (0, i, 0)),
          pl.BlockSpec((ROWS_B, C), lambda i: (i, 0)),
          pl.BlockSpec((NC, ROWS_B, 1), lambda i: (0, i, 0)),
          pl.BlockSpec(Wl.shape, lambda i: (0, 0)),
          pl.BlockSpec(Wr.shape, lambda i: (0, 0)),
          pl.BlockSpec((1, Cz), lambda i: (0, 0)),
      ],
      out_specs=[
          pl.BlockSpec((ROWS_B, Co), lambda i: (i, 0)),
          pl.BlockSpec((ROWS_B, Cz), lambda i: (i, 0)),
      ],
      out_shape=[
          jax.ShapeDtypeStruct((N, Co), jnp.float32),
          jax.ShapeDtypeStruct((N, Cz), jnp.float32),
      ],
  )(p, z, degp, Wl, Wr, b)


def _tc_combine_hz(p, z, degp, Wr, b):
  C = z.shape[1]
  Cz = Wr.shape[0]

  def body(p_ref, z_ref, d_ref, wr_ref, b_ref, h_ref, z2_ref):
    pb = p_ref[...]
    deg = d_ref[0] + d_ref[1]
    dinv = 1.0 / jnp.maximum(deg, 1.0)
    h = jnp.maximum((pb[0] + pb[1]) * dinv + z_ref[...], 0.0)
    h_ref[...] = h
    dn = (((1,), (1,)), ((), ()))
    z2_ref[...] = lax.dot_general(h, wr_ref[...], dn,
                                  preferred_element_type=jnp.float32) + b_ref[...]

  grid = (N // ROWS_B,)
  return pl.pallas_call(
      body,
      grid=grid,
      in_specs=[
          pl.BlockSpec((NC, ROWS_B, C), lambda i: (0, i, 0)),
          pl.BlockSpec((ROWS_B, C), lambda i: (i, 0)),
          pl.BlockSpec((NC, ROWS_B, 1), lambda i: (0, i, 0)),
          pl.BlockSpec(Wr.shape, lambda i: (0, 0)),
          pl.BlockSpec((1, Cz), lambda i: (0, 0)),
      ],
      out_specs=[
          pl.BlockSpec((ROWS_B, C), lambda i: (i, 0)),
          pl.BlockSpec((ROWS_B, Cz), lambda i: (i, 0)),
      ],
      out_shape=[
          jax.ShapeDtypeStruct((N, C), jnp.float32),
          jax.ShapeDtypeStruct((N, Cz), jnp.float32),
      ],
  )(p, z, degp, Wr, b)


def _tc_final(p, z, degp, Wl):
  C = p.shape[2]
  n_out = Wl.shape[0]

  def body(p_ref, z_ref, d_ref, wl_ref, o_ref):
    pb = p_ref[...]
    deg = d_ref[0] + d_ref[1]
    dinv = 1.0 / jnp.maximum(deg, 1.0)
    agg = (pb[0] + pb[1]) * dinv
    dn = (((1,), (1,)), ((), ()))
    h = lax.dot_general(agg, wl_ref[...], dn,
                        preferred_element_type=jnp.float32) + z_ref[...]
    m = jnp.max(h, axis=-1, keepdims=True)
    lse = jnp.log(jnp.sum(jnp.exp(h - m), axis=-1, keepdims=True))
    o_ref[...] = h - m - lse

  grid = (N // ROWS_B,)
  return pl.pallas_call(
      body,
      grid=grid,
      in_specs=[
          pl.BlockSpec((NC, ROWS_B, C), lambda i: (0, i, 0)),
          pl.BlockSpec((ROWS_B, n_out), lambda i: (i, 0)),
          pl.BlockSpec((NC, ROWS_B, 1), lambda i: (0, i, 0)),
          pl.BlockSpec(Wl.shape, lambda i: (0, 0)),
      ],
      out_specs=pl.BlockSpec((ROWS_B, n_out), lambda i: (i, 0)),
      out_shape=jax.ShapeDtypeStruct((N, n_out), jnp.float32),
  )(p, z, degp, Wl)


@jax.jit
def kernel(x, adj_t, Wl0, Wr0, b0, Wl1, Wr1, b1, Wl2, Wr2, b2):
  src2d = adj_t[0].astype(jnp.int32)
  dst2d = adj_t[1].astype(jnp.int32)
  zrows = jnp.zeros((RPT_LAST, 128), jnp.float32)
  zdeg = jnp.zeros((N,), jnp.float32)

  y0, z0 = _tc_matmul2(x, Wl0, Wr0, b0.reshape(1, -1))
  p0, degp = _sc_segment_sum(y0, src2d, dst2d, zrows, zdeg, True)
  degp3 = degp.reshape(NC, N, 1)
  y1, z1 = _tc_combine_matmul2(p0, z0, degp3, Wl1, Wr1, b1.reshape(1, -1))
  (p1,) = _sc_segment_sum(y1, src2d, dst2d, zrows, zdeg, False)
  h2, z2 = _tc_combine_hz(p1, z1, degp3, Wr2, b2.reshape(1, -1))
  (p2,) = _sc_segment_sum(h2, src2d, dst2d, zrows, zdeg, False)
  return _tc_final(p2, z2, degp3, Wl2)


# speedup vs baseline: 14.1629x; 1.0351x over previous
import functools

import jax
import jax.numpy as jnp
from jax import lax
from jax.experimental import pallas as pl
from jax.experimental.pallas import tpu as pltpu
from jax.experimental.pallas import tpu_sc as plsc

N = 10000
E = 320000
NC = 2
NS = 16
NW = NC * NS
ET = E // NW
K = 80
NCHUNK = ET // K
RPT = 624
RPT_LAST = N - RPT * (NS - 1)


def _sc_segment_sum(y, src2d, dst2d, zrows, zdeg, with_deg):
  C = y.shape[1]
  mesh = plsc.VectorSubcoreMesh(core_axis_name="c", subcore_axis_name="s")
  out_type = [jax.ShapeDtypeStruct((NC, N, C), jnp.float32)]
  scratch = [
      pltpu.VMEM_SHARED((N, C), jnp.float32),
      pltpu.VMEM((K,), jnp.int32),
      pltpu.VMEM((K,), jnp.int32),
      pltpu.VMEM((K,), jnp.int32),
      pltpu.VMEM((K,), jnp.int32),
      pltpu.VMEM((K,), jnp.int32),
      pltpu.VMEM((K,), jnp.int32),
      pltpu.VMEM((K,), jnp.int32),
      pltpu.VMEM((K,), jnp.int32),
      pltpu.VMEM((K, C), jnp.float32),
      pltpu.VMEM((K, C), jnp.float32),
      pltpu.VMEM((K, C), jnp.float32),
      pltpu.VMEM((K, C), jnp.float32),
      pltpu.SemaphoreType.DMA,
      pltpu.SemaphoreType.DMA,
      pltpu.SemaphoreType.DMA,
  ]
  if with_deg:
    out_type.append(jax.ShapeDtypeStruct((NC, N), jnp.float32))
    scratch += [
        pltpu.VMEM_SHARED((N,), jnp.float32),
        pltpu.VMEM((K,), jnp.float32),
    ]

  def body(y_h, s_h, d_h, zr_h, zd_h, *rest):
    if with_deg:
      (out_h, deg_h, acc, s0, s1, s2, s3, d0, d1, d2, d3, r0_, r1_, r2_, r3_,
       sem_g, sem_i, sem_s, dacc, ones) = rest
    else:
      (out_h, acc, s0, s1, s2, s3, d0, d1, d2, d3, r0_, r1_, r2_, r3_,
       sem_g, sem_i, sem_s) = rest
    sbufs = (s0, s1, s2, s3)
    dbufs = (d0, d1, d2, d3)
    rbufs = (r0_, r1_, r2_, r3_)
    cid = lax.axis_index("c")
    sid = lax.axis_index("s")
    wid = cid * NS + sid
    r0 = pl.multiple_of(sid * RPT, 8)
    @pl.when(sid < NS - 1)
    def _():
      pltpu.sync_copy(zr_h.at[pl.ds(0, RPT)], acc.at[pl.ds(r0, RPT)])
    @pl.when(sid == NS - 1)
    def _():
      pltpu.sync_copy(zr_h, acc.at[pl.ds(RPT * (NS - 1), RPT_LAST)])
    if with_deg:
      @pl.when(sid == 0)
      def _():
        pltpu.sync_copy(zd_h, dacc)
      for j in range(K // 16):
        ones[pl.ds(j * 16, 16)] = jnp.ones((16,), jnp.float32)
    plsc.subcore_barrier()

    ebase = wid * ET

    def idxload(j, u):
      off = pl.multiple_of(ebase + j * K, 8)
      pltpu.async_copy(s_h.at[pl.ds(off, K)], sbufs[u], sem_i)
      pltpu.async_copy(d_h.at[pl.ds(off, K)], dbufs[u], sem_i)

    def idxwait():
      pltpu.make_async_copy(s_h.at[pl.ds(0, K)], s0, sem_i).wait()
      pltpu.make_async_copy(d_h.at[pl.ds(0, K)], d0, sem_i).wait()

    def gather(u):
      pltpu.async_copy(y_h.at[sbufs[u]], rbufs[u], sem_g)

    def drain_g(u):
      pltpu.make_async_copy(y_h.at[s0], rbufs[u], sem_g).wait()

    def scat(u):
      pltpu.async_copy(rbufs[u], acc.at[dbufs[u]], sem_s, add=True)
      if with_deg:
        pltpu.sync_copy(ones, dacc.at[dbufs[u]], add=True)

    def drain_s(u):
      pltpu.make_async_copy(rbufs[u], acc.at[dbufs[u]], sem_s).wait()

    def half(j, u, first=False, wait_idx=True, do_gather=True, load=True):
      drain_g(u)
      if wait_idx:
        idxwait()
      if not first:
        drain_s((u + 3) % 4)
      if do_gather:
        gather((u + 2) % 4)
      scat(u)
      if load:
        idxload(j + 3, (u + 3) % 4)

    idxload(0, 0)
    idxwait()
    gather(0)
    idxload(1, 1)
    idxwait()
    gather(1)
    idxload(2, 2)
    half(0, 0, first=True)

    def step(i, carry):
      j = 1 + i * 4
      half(j, 1)
      half(j + 1, 2)
      half(j + 2, 3)
      half(j + 3, 0)
      return carry

    lax.fori_loop(0, (NCHUNK - 5) // 4, step, 0)
    half(NCHUNK - 4, 1)
    half(NCHUNK - 3, 2, load=False)
    half(NCHUNK - 2, 3, wait_idx=False, do_gather=False, load=False)
    half(NCHUNK - 1, 0, wait_idx=False, do_gather=False, load=False)
    drain_s(0)
    plsc.subcore_barrier()
    @pl.when(sid < NS - 1)
    def _():
      pltpu.sync_copy(acc.at[pl.ds(r0, RPT)], out_h.at[cid, pl.ds(r0, RPT)])
    @pl.when(sid == NS - 1)
    def _():
      last = RPT * (NS - 1)
      pltpu.sync_copy(acc.at[pl.ds(last, RPT_LAST)],
                      out_h.at[cid, pl.ds(last, RPT_LAST)])
    if with_deg:
      @pl.when(sid == 0)
      def _():
        pltpu.sync_copy(dacc, deg_h.at[cid])

  fn = pl.kernel(body, out_type=out_type, mesh=mesh, scratch_types=scratch)
  return fn(y, src2d, dst2d, zrows, zdeg)


ROWS_B = 2000


def _tc_matmul2(x, Wl, Wr, b, adj):
  Co = Wl.shape[0]
  Cz = Wr.shape[0]

  def body(x_ref, wl_ref, wr_ref, b_ref, a_ref, y_ref, z_ref, s_ref, d_ref):
    xb = x_ref[...]
    dn = (((1,), (1,)), ((), ()))
    y_ref[...] = lax.dot_general(xb, wl_ref[...], dn,
                                 preferred_element_type=jnp.float32)
    z_ref[...] = lax.dot_general(xb, wr_ref[...], dn,
                                 preferred_element_type=jnp.float32) + b_ref[...]
    @pl.when(pl.program_id(0) == 0)
    def _():
      s_ref[...] = a_ref[0]
      d_ref[...] = a_ref[1]

  grid = (N // ROWS_B,)
  return pl.pallas_call(
      body,
      grid=grid,
      in_specs=[
          pl.BlockSpec((ROWS_B, x.shape[1]), lambda i: (i, 0)),
          pl.BlockSpec(Wl.shape, lambda i: (0, 0)),
          pl.BlockSpec(Wr.shape, lambda i: (0, 0)),
          pl.BlockSpec((1, Cz), lambda i: (0, 0)),
          pl.BlockSpec((2, E), lambda i: (0, 0)),
      ],
      out_specs=[
          pl.BlockSpec((ROWS_B, Co), lambda i: (i, 0)),
          pl.BlockSpec((ROWS_B, Cz), lambda i: (i, 0)),
          pl.BlockSpec((E,), lambda i: (0,)),
          pl.BlockSpec((E,), lambda i: (0,)),
      ],
      out_shape=[
          jax.ShapeDtypeStruct((N, Co), jnp.float32),
          jax.ShapeDtypeStruct((N, Cz), jnp.float32),
          jax.ShapeDtypeStruct((E,), jnp.int32),
          jax.ShapeDtypeStruct((E,), jnp.int32),
      ],
  )(x, Wl, Wr, b, adj)


def _tc_combine_matmul2(p, z, degp, Wl, Wr, b):
  C = z.shape[1]
  Co = Wl.shape[0]
  Cz = Wr.shape[0]

  def body(p_ref, z_ref, d_ref, wl_ref, wr_ref, b_ref, y_ref, z2_ref):
    pb = p_ref[...]
    deg = d_ref[0] + d_ref[1]
    dinv = 1.0 / jnp.maximum(deg, 1.0)
    h = jnp.maximum((pb[0] + pb[1]) * dinv + z_ref[...], 0.0)
    dn = (((1,), (1,)), ((), ()))
    y_ref[...] = lax.dot_general(h, wl_ref[...], dn,
                                 preferred_element_type=jnp.float32)
    z2_ref[...] = lax.dot_general(h, wr_ref[...], dn,
                                  preferred_element_type=jnp.float32) + b_ref[...]

  grid = (N // ROWS_B,)
  return pl.pallas_call(
      body,
      grid=grid,
      in_specs=[
          pl.BlockSpec((NC, ROWS_B, C), lambda i: (0, i, 0)),
          pl.BlockSpec((ROWS_B, C), lambda i: (i, 0)),
          pl.BlockSpec((NC, ROWS_B, 1), lambda i: (0, i, 0)),
          pl.BlockSpec(Wl.shape, lambda i: (0, 0)),
          pl.BlockSpec(Wr.shape, lambda i: (0, 0)),
          pl.BlockSpec((1, Cz), lambda i: (0, 0)),
      ],
      out_specs=[
          pl.BlockSpec((ROWS_B, Co), lambda i: (i, 0)),
          pl.BlockSpec((ROWS_B, Cz), lambda i: (i, 0)),
      ],
      out_shape=[
          jax.ShapeDtypeStruct((N, Co), jnp.float32),
          jax.ShapeDtypeStruct((N, Cz), jnp.float32),
      ],
  )(p, z, degp, Wl, Wr, b)


def _tc_combine_hz(p, z, degp, Wr, b):
  C = z.shape[1]
  Cz = Wr.shape[0]

  def body(p_ref, z_ref, d_ref, wr_ref, b_ref, h_ref, z2_ref):
    pb = p_ref[...]
    deg = d_ref[0] + d_ref[1]
    dinv = 1.0 / jnp.maximum(deg, 1.0)
    h = jnp.maximum((pb[0] + pb[1]) * dinv + z_ref[...], 0.0)
    h_ref[...] = h
    dn = (((1,), (1,)), ((), ()))
    z2_ref[...] = lax.dot_general(h, wr_ref[...], dn,
                                  preferred_element_type=jnp.float32) + b_ref[...]

  grid = (N // ROWS_B,)
  return pl.pallas_call(
      body,
      grid=grid,
      in_specs=[
          pl.BlockSpec((NC, ROWS_B, C), lambda i: (0, i, 0)),
          pl.BlockSpec((ROWS_B, C), lambda i: (i, 0)),
          pl.BlockSpec((NC, ROWS_B, 1), lambda i: (0, i, 0)),
          pl.BlockSpec(Wr.shape, lambda i: (0, 0)),
          pl.BlockSpec((1, Cz), lambda i: (0, 0)),
      ],
      out_specs=[
          pl.BlockSpec((ROWS_B, C), lambda i: (i, 0)),
          pl.BlockSpec((ROWS_B, Cz), lambda i: (i, 0)),
      ],
      out_shape=[
          jax.ShapeDtypeStruct((N, C), jnp.float32),
          jax.ShapeDtypeStruct((N, Cz), jnp.float32),
      ],
  )(p, z, degp, Wr, b)


def _tc_final(p, z, degp, Wl):
  C = p.shape[2]
  n_out = Wl.shape[0]

  def body(p_ref, z_ref, d_ref, wl_ref, o_ref):
    pb = p_ref[...]
    deg = d_ref[0] + d_ref[1]
    dinv = 1.0 / jnp.maximum(deg, 1.0)
    agg = (pb[0] + pb[1]) * dinv
    dn = (((1,), (1,)), ((), ()))
    h = lax.dot_general(agg, wl_ref[...], dn,
                        preferred_element_type=jnp.float32) + z_ref[...]
    m = jnp.max(h, axis=-1, keepdims=True)
    lse = jnp.log(jnp.sum(jnp.exp(h - m), axis=-1, keepdims=True))
    o_ref[...] = h - m - lse

  grid = (N // ROWS_B,)
  return pl.pallas_call(
      body,
      grid=grid,
      in_specs=[
          pl.BlockSpec((NC, ROWS_B, C), lambda i: (0, i, 0)),
          pl.BlockSpec((ROWS_B, n_out), lambda i: (i, 0)),
          pl.BlockSpec((NC, ROWS_B, 1), lambda i: (0, i, 0)),
          pl.BlockSpec(Wl.shape, lambda i: (0, 0)),
      ],
      out_specs=pl.BlockSpec((ROWS_B, n_out), lambda i: (i, 0)),
      out_shape=jax.ShapeDtypeStruct((N, n_out), jnp.float32),
  )(p, z, degp, Wl)


@jax.jit
def kernel(x, adj_t, Wl0, Wr0, b0, Wl1, Wr1, b1, Wl2, Wr2, b2):
  adj = adj_t.astype(jnp.int32)
  zrows = jnp.zeros((RPT_LAST, 128), jnp.float32)
  zdeg = jnp.zeros((N,), jnp.float32)

  y0, z0, src, dst = _tc_matmul2(x, Wl0, Wr0, b0.reshape(1, -1), adj)
  p0, degp = _sc_segment_sum(y0, src, dst, zrows, zdeg, True)
  degp3 = degp.reshape(NC, N, 1)
  y1, z1 = _tc_combine_matmul2(p0, z0, degp3, Wl1, Wr1, b1.reshape(1, -1))
  (p1,) = _sc_segment_sum(y1, src, dst, zrows, zdeg, False)
  h2, z2 = _tc_combine_hz(p1, z1, degp3, Wr2, b2.reshape(1, -1))
  (p2,) = _sc_segment_sum(h2, src, dst, zrows, zdeg, False)
  return _tc_final(p2, z2, degp3, Wl2)


# speedup vs baseline: 14.6393x; 1.0336x over previous
import functools

import jax
import jax.numpy as jnp
from jax import lax
from jax.experimental import pallas as pl
from jax.experimental.pallas import tpu as pltpu
from jax.experimental.pallas import tpu_sc as plsc

N = 10000
E = 320000
NC = 2
NS = 16
NW = NC * NS
ET = E // NW
K = 80
NCHUNK = ET // K
RPT = 624
RPT_LAST = N - RPT * (NS - 1)


def _sc_segment_sum(y, src2d, dst2d, zrows, zdeg, with_deg):
  C = y.shape[1]
  mesh = plsc.VectorSubcoreMesh(core_axis_name="c", subcore_axis_name="s")
  out_type = [jax.ShapeDtypeStruct((NC, N, C), jnp.float32)]
  scratch = [
      pltpu.VMEM_SHARED((N, C), jnp.float32),
      pltpu.VMEM((K,), jnp.int32),
      pltpu.VMEM((K,), jnp.int32),
      pltpu.VMEM((K,), jnp.int32),
      pltpu.VMEM((K,), jnp.int32),
      pltpu.VMEM((K,), jnp.int32),
      pltpu.VMEM((K,), jnp.int32),
      pltpu.VMEM((K,), jnp.int32),
      pltpu.VMEM((K,), jnp.int32),
      pltpu.VMEM((K, C), jnp.float32),
      pltpu.VMEM((K, C), jnp.float32),
      pltpu.VMEM((K, C), jnp.float32),
      pltpu.VMEM((K, C), jnp.float32),
      pltpu.SemaphoreType.DMA,
      pltpu.SemaphoreType.DMA,
      pltpu.SemaphoreType.DMA,
  ]
  if with_deg:
    out_type.append(jax.ShapeDtypeStruct((NC, N), jnp.float32))
    scratch += [
        pltpu.VMEM_SHARED((N,), jnp.float32),
        pltpu.VMEM((K,), jnp.float32),
    ]

  def body(y_h, s_h, d_h, zr_h, zd_h, *rest):
    if with_deg:
      (out_h, deg_h, acc, s0, s1, s2, s3, d0, d1, d2, d3, r0_, r1_, r2_, r3_,
       sem_g, sem_i, sem_s, dacc, ones) = rest
    else:
      (out_h, acc, s0, s1, s2, s3, d0, d1, d2, d3, r0_, r1_, r2_, r3_,
       sem_g, sem_i, sem_s) = rest
    sbufs = (s0, s1, s2, s3)
    dbufs = (d0, d1, d2, d3)
    rbufs = (r0_, r1_, r2_, r3_)
    cid = lax.axis_index("c")
    sid = lax.axis_index("s")
    wid = cid * NS + sid
    r0 = pl.multiple_of(sid * RPT, 8)
    @pl.when(sid < NS - 1)
    def _():
      pltpu.sync_copy(zr_h.at[pl.ds(0, RPT)], acc.at[pl.ds(r0, RPT)])
    @pl.when(sid == NS - 1)
    def _():
      pltpu.sync_copy(zr_h, acc.at[pl.ds(RPT * (NS - 1), RPT_LAST)])
    if with_deg:
      @pl.when(sid == 0)
      def _():
        pltpu.sync_copy(zd_h, dacc)
      for j in range(K // 16):
        ones[pl.ds(j * 16, 16)] = jnp.ones((16,), jnp.float32)
    plsc.subcore_barrier()

    ebase = wid * ET

    def idxload(j, u):
      off = pl.multiple_of(ebase + j * K, 8)
      pltpu.async_copy(s_h.at[pl.ds(off, K)], sbufs[u], sem_i)
      pltpu.async_copy(d_h.at[pl.ds(off, K)], dbufs[u], sem_i)

    def idxwait():
      pltpu.make_async_copy(s_h.at[pl.ds(0, K)], s0, sem_i).wait()
      pltpu.make_async_copy(d_h.at[pl.ds(0, K)], d0, sem_i).wait()

    def gather(u):
      pltpu.async_copy(y_h.at[sbufs[u]], rbufs[u], sem_g)

    def drain_g(u):
      pltpu.make_async_copy(y_h.at[s0], rbufs[u], sem_g).wait()

    def scat(u):
      pltpu.async_copy(rbufs[u], acc.at[dbufs[u]], sem_s, add=True)
      if with_deg:
        pltpu.sync_copy(ones, dacc.at[dbufs[u]], add=True)

    def drain_s(u):
      pltpu.make_async_copy(rbufs[u], acc.at[dbufs[u]], sem_s).wait()

    def half(j, u, first=False, wait_idx=True, do_gather=True, load=True):
      drain_g(u)
      if wait_idx:
        idxwait()
      if not first:
        drain_s((u + 3) % 4)
      if do_gather:
        gather((u + 2) % 4)
      scat(u)
      if load:
        idxload(j + 3, (u + 3) % 4)

    idxload(0, 0)
    idxwait()
    gather(0)
    idxload(1, 1)
    idxwait()
    gather(1)
    idxload(2, 2)
    half(0, 0, first=True)

    def step(i, carry):
      j = 1 + i * 4
      half(j, 1)
      half(j + 1, 2)
      half(j + 2, 3)
      half(j + 3, 0)
      return carry

    lax.fori_loop(0, (NCHUNK - 5) // 4, step, 0)
    half(NCHUNK - 4, 1)
    half(NCHUNK - 3, 2, load=False)
    half(NCHUNK - 2, 3, wait_idx=False, do_gather=False, load=False)
    half(NCHUNK - 1, 0, wait_idx=False, do_gather=False, load=False)
    drain_s(0)
    plsc.subcore_barrier()
    @pl.when(sid < NS - 1)
    def _():
      pltpu.sync_copy(acc.at[pl.ds(r0, RPT)], out_h.at[cid, pl.ds(r0, RPT)])
    @pl.when(sid == NS - 1)
    def _():
      last = RPT * (NS - 1)
      pltpu.sync_copy(acc.at[pl.ds(last, RPT_LAST)],
                      out_h.at[cid, pl.ds(last, RPT_LAST)])
    if with_deg:
      @pl.when(sid == 0)
      def _():
        pltpu.sync_copy(dacc, deg_h.at[cid])

  fn = pl.kernel(body, out_type=out_type, mesh=mesh, scratch_types=scratch)
  return fn(y, src2d, dst2d, zrows, zdeg)


ROWS_B = 2048


def _tc_matmul2(x, Wl, Wr, b, adj):
  Co = Wl.shape[0]
  Cz = Wr.shape[0]

  def body(x_ref, wl_ref, wr_ref, b_ref, a_ref, y_ref, z_ref, s_ref, d_ref):
    xb = x_ref[...]
    dn = (((1,), (1,)), ((), ()))
    y_ref[...] = lax.dot_general(xb, wl_ref[...], dn,
                                 preferred_element_type=jnp.float32)
    z_ref[...] = lax.dot_general(xb, wr_ref[...], dn,
                                 preferred_element_type=jnp.float32) + b_ref[...]
    @pl.when(pl.program_id(0) == 0)
    def _():
      s_ref[...] = a_ref[0]
      d_ref[...] = a_ref[1]

  grid = (pl.cdiv(N, ROWS_B),)
  return pl.pallas_call(
      body,
      grid=grid,
      in_specs=[
          pl.BlockSpec((ROWS_B, x.shape[1]), lambda i: (i, 0)),
          pl.BlockSpec(Wl.shape, lambda i: (0, 0)),
          pl.BlockSpec(Wr.shape, lambda i: (0, 0)),
          pl.BlockSpec((1, Cz), lambda i: (0, 0)),
          pl.BlockSpec((2, E), lambda i: (0, 0)),
      ],
      out_specs=[
          pl.BlockSpec((ROWS_B, Co), lambda i: (i, 0)),
          pl.BlockSpec((ROWS_B, Cz), lambda i: (i, 0)),
          pl.BlockSpec((E,), lambda i: (0,)),
          pl.BlockSpec((E,), lambda i: (0,)),
      ],
      out_shape=[
          jax.ShapeDtypeStruct((N, Co), jnp.float32),
          jax.ShapeDtypeStruct((N, Cz), jnp.float32),
          jax.ShapeDtypeStruct((E,), jnp.int32),
          jax.ShapeDtypeStruct((E,), jnp.int32),
      ],
  )(x, Wl, Wr, b, adj)


def _tc_combine_matmul2(p, z, degp, Wl, Wr, b):
  C = z.shape[1]
  Co = Wl.shape[0]
  Cz = Wr.shape[0]

  def body(p_ref, z_ref, d_ref, wl_ref, wr_ref, b_ref, y_ref, z2_ref):
    pb = p_ref[...]
    deg = d_ref[0] + d_ref[1]
    dinv = (1.0 / jnp.maximum(deg, 1.0))[:, None]
    h = jnp.maximum((pb[0] + pb[1]) * dinv + z_ref[...], 0.0)
    dn = (((1,), (1,)), ((), ()))
    y_ref[...] = lax.dot_general(h, wl_ref[...], dn,
                                 preferred_element_type=jnp.float32)
    z2_ref[...] = lax.dot_general(h, wr_ref[...], dn,
                                  preferred_element_type=jnp.float32) + b_ref[...]

  grid = (pl.cdiv(N, ROWS_B),)
  return pl.pallas_call(
      body,
      grid=grid,
      in_specs=[
          pl.BlockSpec((NC, ROWS_B, C), lambda i: (0, i, 0)),
          pl.BlockSpec((ROWS_B, C), lambda i: (i, 0)),
          pl.BlockSpec((NC, ROWS_B), lambda i: (0, i)),
          pl.BlockSpec(Wl.shape, lambda i: (0, 0)),
          pl.BlockSpec(Wr.shape, lambda i: (0, 0)),
          pl.BlockSpec((1, Cz), lambda i: (0, 0)),
      ],
      out_specs=[
          pl.BlockSpec((ROWS_B, Co), lambda i: (i, 0)),
          pl.BlockSpec((ROWS_B, Cz), lambda i: (i, 0)),
      ],
      out_shape=[
          jax.ShapeDtypeStruct((N, Co), jnp.float32),
          jax.ShapeDtypeStruct((N, Cz), jnp.float32),
      ],
  )(p, z, degp, Wl, Wr, b)


def _tc_combine_hz(p, z, degp, Wr, b):
  C = z.shape[1]
  Cz = Wr.shape[0]

  def body(p_ref, z_ref, d_ref, wr_ref, b_ref, h_ref, z2_ref):
    pb = p_ref[...]
    deg = d_ref[0] + d_ref[1]
    dinv = (1.0 / jnp.maximum(deg, 1.0))[:, None]
    h = jnp.maximum((pb[0] + pb[1]) * dinv + z_ref[...], 0.0)
    h_ref[...] = h
    dn = (((1,), (1,)), ((), ()))
    z2_ref[...] = lax.dot_general(h, wr_ref[...], dn,
                                  preferred_element_type=jnp.float32) + b_ref[...]

  grid = (pl.cdiv(N, ROWS_B),)
  return pl.pallas_call(
      body,
      grid=grid,
      in_specs=[
          pl.BlockSpec((NC, ROWS_B, C), lambda i: (0, i, 0)),
          pl.BlockSpec((ROWS_B, C), lambda i: (i, 0)),
          pl.BlockSpec((NC, ROWS_B), lambda i: (0, i)),
          pl.BlockSpec(Wr.shape, lambda i: (0, 0)),
          pl.BlockSpec((1, Cz), lambda i: (0, 0)),
      ],
      out_specs=[
          pl.BlockSpec((ROWS_B, C), lambda i: (i, 0)),
          pl.BlockSpec((ROWS_B, Cz), lambda i: (i, 0)),
      ],
      out_shape=[
          jax.ShapeDtypeStruct((N, C), jnp.float32),
          jax.ShapeDtypeStruct((N, Cz), jnp.float32),
      ],
  )(p, z, degp, Wr, b)


def _tc_final(p, z, degp, Wl):
  C = p.shape[2]
  n_out = Wl.shape[0]

  def body(p_ref, z_ref, d_ref, wl_ref, o_ref):
    pb = p_ref[...]
    deg = d_ref[0] + d_ref[1]
    dinv = (1.0 / jnp.maximum(deg, 1.0))[:, None]
    agg = (pb[0] + pb[1]) * dinv
    dn = (((1,), (1,)), ((), ()))
    h = lax.dot_general(agg, wl_ref[...], dn,
                        preferred_element_type=jnp.float32) + z_ref[...]
    m = jnp.max(h, axis=-1, keepdims=True)
    lse = jnp.log(jnp.sum(jnp.exp(h - m), axis=-1, keepdims=True))
    o_ref[...] = h - m - lse

  grid = (pl.cdiv(N, ROWS_B),)
  return pl.pallas_call(
      body,
      grid=grid,
      in_specs=[
          pl.BlockSpec((NC, ROWS_B, C), lambda i: (0, i, 0)),
          pl.BlockSpec((ROWS_B, n_out), lambda i: (i, 0)),
          pl.BlockSpec((NC, ROWS_B), lambda i: (0, i)),
          pl.BlockSpec(Wl.shape, lambda i: (0, 0)),
      ],
      out_specs=pl.BlockSpec((ROWS_B, n_out), lambda i: (i, 0)),
      out_shape=jax.ShapeDtypeStruct((N, n_out), jnp.float32),
  )(p, z, degp, Wl)


@jax.jit
def kernel(x, adj_t, Wl0, Wr0, b0, Wl1, Wr1, b1, Wl2, Wr2, b2):
  adj = adj_t.astype(jnp.int32)
  zrows = jnp.zeros((RPT_LAST, 128), jnp.float32)
  zdeg = jnp.zeros((N,), jnp.float32)

  y0, z0, src, dst = _tc_matmul2(x, Wl0, Wr0, b0.reshape(1, -1), adj)
  p0, degp = _sc_segment_sum(y0, src, dst, zrows, zdeg, True)
  y1, z1 = _tc_combine_matmul2(p0, z0, degp, Wl1, Wr1, b1.reshape(1, -1))
  (p1,) = _sc_segment_sum(y1, src, dst, zrows, zdeg, False)
  h2, z2 = _tc_combine_hz(p1, z1, degp, Wr2, b2.reshape(1, -1))
  (p2,) = _sc_segment_sum(h2, src, dst, zrows, zdeg, False)
  return _tc_final(p2, z2, degp, Wl2)


# speedup vs baseline: 14.8464x; 1.0142x over previous
import functools

import jax
import jax.numpy as jnp
from jax import lax
from jax.experimental import pallas as pl
from jax.experimental.pallas import tpu as pltpu
from jax.experimental.pallas import tpu_sc as plsc

N = 10000
E = 320000
NC = 2
NS = 16
NW = NC * NS
ET = E // NW
K = 80
NCHUNK = ET // K
RPT = 624
RPT_LAST = N - RPT * (NS - 1)


def _sc_segment_sum(y, src2d, dst2d, zrows, zdeg, with_deg):
  C = y.shape[1]
  mesh = plsc.VectorSubcoreMesh(core_axis_name="c", subcore_axis_name="s")
  out_type = [jax.ShapeDtypeStruct((NC, N, C), jnp.float32)]
  scratch = [
      pltpu.VMEM_SHARED((N, C), jnp.float32),
      pltpu.VMEM((K,), jnp.int32),
      pltpu.VMEM((K,), jnp.int32),
      pltpu.VMEM((K,), jnp.int32),
      pltpu.VMEM((K,), jnp.int32),
      pltpu.VMEM((K,), jnp.int32),
      pltpu.VMEM((K,), jnp.int32),
      pltpu.VMEM((K,), jnp.int32),
      pltpu.VMEM((K,), jnp.int32),
      pltpu.VMEM((K, C), jnp.float32),
      pltpu.VMEM((K, C), jnp.float32),
      pltpu.VMEM((K, C), jnp.float32),
      pltpu.VMEM((K, C), jnp.float32),
      pltpu.SemaphoreType.DMA,
      pltpu.SemaphoreType.DMA,
      pltpu.SemaphoreType.DMA,
      pltpu.SemaphoreType.DMA,
  ]
  if with_deg:
    out_type.append(jax.ShapeDtypeStruct((NC, N), jnp.float32))
    scratch += [
        pltpu.VMEM_SHARED((N,), jnp.float32),
        pltpu.VMEM((K,), jnp.float32),
    ]

  def body(y_h, s_h, d_h, zr_h, zd_h, *rest):
    if with_deg:
      (out_h, deg_h, acc, s0, s1, s2, s3, d0, d1, d2, d3, r0_, r1_, r2_, r3_,
       sem_g, sem_i, sem_s, sem_o, dacc, ones) = rest
    else:
      (out_h, acc, s0, s1, s2, s3, d0, d1, d2, d3, r0_, r1_, r2_, r3_,
       sem_g, sem_i, sem_s, sem_o) = rest
    sbufs = (s0, s1, s2, s3)
    dbufs = (d0, d1, d2, d3)
    rbufs = (r0_, r1_, r2_, r3_)
    cid = lax.axis_index("c")
    sid = lax.axis_index("s")
    wid = cid * NS + sid
    r0 = pl.multiple_of(sid * RPT, 8)
    @pl.when(sid < NS - 1)
    def _():
      pltpu.sync_copy(zr_h.at[pl.ds(0, RPT)], acc.at[pl.ds(r0, RPT)])
    @pl.when(sid == NS - 1)
    def _():
      pltpu.sync_copy(zr_h, acc.at[pl.ds(RPT * (NS - 1), RPT_LAST)])
    if with_deg:
      @pl.when(sid == 0)
      def _():
        pltpu.sync_copy(zd_h, dacc)
      for j in range(K // 16):
        ones[pl.ds(j * 16, 16)] = jnp.ones((16,), jnp.float32)
    plsc.subcore_barrier()

    ebase = wid * ET

    def idxload(j, u):
      off = pl.multiple_of(ebase + j * K, 8)
      pltpu.async_copy(s_h.at[pl.ds(off, K)], sbufs[u], sem_i)
      pltpu.async_copy(d_h.at[pl.ds(off, K)], dbufs[u], sem_i)

    def idxwait():
      pltpu.make_async_copy(s_h.at[pl.ds(0, K)], s0, sem_i).wait()
      pltpu.make_async_copy(d_h.at[pl.ds(0, K)], d0, sem_i).wait()

    def gather(u):
      pltpu.async_copy(y_h.at[sbufs[u]], rbufs[u], sem_g)

    def drain_g(u):
      pltpu.make_async_copy(y_h.at[s0], rbufs[u], sem_g).wait()

    def scat(u):
      pltpu.async_copy(rbufs[u], acc.at[dbufs[u]], sem_s, add=True)
      if with_deg:
        pltpu.async_copy(ones, dacc.at[dbufs[u]], sem_o, add=True)

    def drain_s(u):
      pltpu.make_async_copy(rbufs[u], acc.at[dbufs[u]], sem_s).wait()
      if with_deg:
        pltpu.make_async_copy(ones, dacc.at[dbufs[u]], sem_o).wait()

    def half(j, u, first=False, wait_idx=True, do_gather=True, load=True):
      drain_g(u)
      if wait_idx:
        idxwait()
      if not first:
        drain_s((u + 3) % 4)
      if do_gather:
        gather((u + 2) % 4)
      scat(u)
      if load:
        idxload(j + 3, (u + 3) % 4)

    idxload(0, 0)
    idxwait()
    gather(0)
    idxload(1, 1)
    idxwait()
    gather(1)
    idxload(2, 2)
    half(0, 0, first=True)

    def step(i, carry):
      j = 1 + i * 4
      half(j, 1)
      half(j + 1, 2)
      half(j + 2, 3)
      half(j + 3, 0)
      return carry

    lax.fori_loop(0, (NCHUNK - 5) // 4, step, 0)
    half(NCHUNK - 4, 1)
    half(NCHUNK - 3, 2, load=False)
    half(NCHUNK - 2, 3, wait_idx=False, do_gather=False, load=False)
    half(NCHUNK - 1, 0, wait_idx=False, do_gather=False, load=False)
    drain_s(0)
    plsc.subcore_barrier()
    @pl.when(sid < NS - 1)
    def _():
      pltpu.sync_copy(acc.at[pl.ds(r0, RPT)], out_h.at[cid, pl.ds(r0, RPT)])
    @pl.when(sid == NS - 1)
    def _():
      last = RPT * (NS - 1)
      pltpu.sync_copy(acc.at[pl.ds(last, RPT_LAST)],
                      out_h.at[cid, pl.ds(last, RPT_LAST)])
    if with_deg:
      @pl.when(sid == 0)
      def _():
        pltpu.sync_copy(dacc, deg_h.at[cid])

  fn = pl.kernel(body, out_type=out_type, mesh=mesh, scratch_types=scratch)
  return fn(y, src2d, dst2d, zrows, zdeg)


ROWS_B = 2048


def _tc_matmul2(x, Wl, Wr, b, adj):
  Co = Wl.shape[0]
  Cz = Wr.shape[0]

  def body(x_ref, wl_ref, wr_ref, b_ref, a_ref, y_ref, z_ref, s_ref, d_ref):
    xb = x_ref[...]
    dn = (((1,), (1,)), ((), ()))
    y_ref[...] = lax.dot_general(xb, wl_ref[...], dn,
                                 preferred_element_type=jnp.float32)
    z_ref[...] = lax.dot_general(xb, wr_ref[...], dn,
                                 preferred_element_type=jnp.float32) + b_ref[...]
    @pl.when(pl.program_id(0) == 0)
    def _():
      s_ref[...] = a_ref[0]
      d_ref[...] = a_ref[1]

  grid = (pl.cdiv(N, ROWS_B),)
  return pl.pallas_call(
      body,
      grid=grid,
      in_specs=[
          pl.BlockSpec((ROWS_B, x.shape[1]), lambda i: (i, 0)),
          pl.BlockSpec(Wl.shape, lambda i: (0, 0)),
          pl.BlockSpec(Wr.shape, lambda i: (0, 0)),
          pl.BlockSpec((1, Cz), lambda i: (0, 0)),
          pl.BlockSpec((2, E), lambda i: (0, 0)),
      ],
      out_specs=[
          pl.BlockSpec((ROWS_B, Co), lambda i: (i, 0)),
          pl.BlockSpec((ROWS_B, Cz), lambda i: (i, 0)),
          pl.BlockSpec((E,), lambda i: (0,)),
          pl.BlockSpec((E,), lambda i: (0,)),
      ],
      out_shape=[
          jax.ShapeDtypeStruct((N, Co), jnp.float32),
          jax.ShapeDtypeStruct((N, Cz), jnp.float32),
          jax.ShapeDtypeStruct((E,), jnp.int32),
          jax.ShapeDtypeStruct((E,), jnp.int32),
      ],
  )(x, Wl, Wr, b, adj)


def _tc_combine_matmul2(p, z, degp, Wl, Wr, b):
  C = z.shape[1]
  Co = Wl.shape[0]
  Cz = Wr.shape[0]

  def body(p_ref, z_ref, d_ref, wl_ref, wr_ref, b_ref, y_ref, z2_ref):
    pb = p_ref[...]
    deg = d_ref[0] + d_ref[1]
    dinv = (1.0 / jnp.maximum(deg, 1.0))[:, None]
    h = jnp.maximum((pb[0] + pb[1]) * dinv + z_ref[...], 0.0)
    dn = (((1,), (1,)), ((), ()))
    y_ref[...] = lax.dot_general(h, wl_ref[...], dn,
                                 preferred_element_type=jnp.float32)
    z2_ref[...] = lax.dot_general(h, wr_ref[...], dn,
                                  preferred_element_type=jnp.float32) + b_ref[...]

  grid = (pl.cdiv(N, ROWS_B),)
  return pl.pallas_call(
      body,
      grid=grid,
      in_specs=[
          pl.BlockSpec((NC, ROWS_B, C), lambda i: (0, i, 0)),
          pl.BlockSpec((ROWS_B, C), lambda i: (i, 0)),
          pl.BlockSpec((NC, ROWS_B), lambda i: (0, i)),
          pl.BlockSpec(Wl.shape, lambda i: (0, 0)),
          pl.BlockSpec(Wr.shape, lambda i: (0, 0)),
          pl.BlockSpec((1, Cz), lambda i: (0, 0)),
      ],
      out_specs=[
          pl.BlockSpec((ROWS_B, Co), lambda i: (i, 0)),
          pl.BlockSpec((ROWS_B, Cz), lambda i: (i, 0)),
      ],
      out_shape=[
          jax.ShapeDtypeStruct((N, Co), jnp.float32),
          jax.ShapeDtypeStruct((N, Cz), jnp.float32),
      ],
  )(p, z, degp, Wl, Wr, b)


def _tc_combine_hz(p, z, degp, Wr, b):
  C = z.shape[1]
  Cz = Wr.shape[0]

  def body(p_ref, z_ref, d_ref, wr_ref, b_ref, h_ref, z2_ref):
    pb = p_ref[...]
    deg = d_ref[0] + d_ref[1]
    dinv = (1.0 / jnp.maximum(deg, 1.0))[:, None]
    h = jnp.maximum((pb[0] + pb[1]) * dinv + z_ref[...], 0.0)
    h_ref[...] = h
    dn = (((1,), (1,)), ((), ()))
    z2_ref[...] = lax.dot_general(h, wr_ref[...], dn,
                                  preferred_element_type=jnp.float32) + b_ref[...]

  grid = (pl.cdiv(N, ROWS_B),)
  return pl.pallas_call(
      body,
      grid=grid,
      in_specs=[
          pl.BlockSpec((NC, ROWS_B, C), lambda i: (0, i, 0)),
          pl.BlockSpec((ROWS_B, C), lambda i: (i, 0)),
          pl.BlockSpec((NC, ROWS_B), lambda i: (0, i)),
          pl.BlockSpec(Wr.shape, lambda i: (0, 0)),
          pl.BlockSpec((1, Cz), lambda i: (0, 0)),
      ],
      out_specs=[
          pl.BlockSpec((ROWS_B, C), lambda i: (i, 0)),
          pl.BlockSpec((ROWS_B, Cz), lambda i: (i, 0)),
      ],
      out_shape=[
          jax.ShapeDtypeStruct((N, C), jnp.float32),
          jax.ShapeDtypeStruct((N, Cz), jnp.float32),
      ],
  )(p, z, degp, Wr, b)


def _tc_final(p, z, degp, Wl):
  C = p.shape[2]
  n_out = Wl.shape[0]

  def body(p_ref, z_ref, d_ref, wl_ref, o_ref):
    pb = p_ref[...]
    deg = d_ref[0] + d_ref[1]
    dinv = (1.0 / jnp.maximum(deg, 1.0))[:, None]
    agg = (pb[0] + pb[1]) * dinv
    dn = (((1,), (1,)), ((), ()))
    h = lax.dot_general(agg, wl_ref[...], dn,
                        preferred_element_type=jnp.float32) + z_ref[...]
    m = jnp.max(h, axis=-1, keepdims=True)
    lse = jnp.log(jnp.sum(jnp.exp(h - m), axis=-1, keepdims=True))
    o_ref[...] = h - m - lse

  grid = (pl.cdiv(N, ROWS_B),)
  return pl.pallas_call(
      body,
      grid=grid,
      in_specs=[
          pl.BlockSpec((NC, ROWS_B, C), lambda i: (0, i, 0)),
          pl.BlockSpec((ROWS_B, n_out), lambda i: (i, 0)),
          pl.BlockSpec((NC, ROWS_B), lambda i: (0, i)),
          pl.BlockSpec(Wl.shape, lambda i: (0, 0)),
      ],
      out_specs=pl.BlockSpec((ROWS_B, n_out), lambda i: (i, 0)),
      out_shape=jax.ShapeDtypeStruct((N, n_out), jnp.float32),
  )(p, z, degp, Wl)


@jax.jit
def kernel(x, adj_t, Wl0, Wr0, b0, Wl1, Wr1, b1, Wl2, Wr2, b2):
  adj = adj_t.astype(jnp.int32)
  zrows = jnp.zeros((RPT_LAST, 128), jnp.float32)
  zdeg = jnp.zeros((N,), jnp.float32)

  y0, z0, src, dst = _tc_matmul2(x, Wl0, Wr0, b0.reshape(1, -1), adj)
  p0, degp = _sc_segment_sum(y0, src, dst, zrows, zdeg, True)
  y1, z1 = _tc_combine_matmul2(p0, z0, degp, Wl1, Wr1, b1.reshape(1, -1))
  (p1,) = _sc_segment_sum(y1, src, dst, zrows, zdeg, False)
  h2, z2 = _tc_combine_hz(p1, z1, degp, Wr2, b2.reshape(1, -1))
  (p2,) = _sc_segment_sum(h2, src, dst, zrows, zdeg, False)
  return _tc_final(p2, z2, degp, Wl2)
